# Initial kernel scaffold; baseline (speedup 1.0000x reference)
import jax
import jax.numpy as jnp
from jax.experimental import pallas as pl


def kernel(x, edge_attr, params, edge_index, batch):
    raise NotImplementedError("write your pallas kernel here")


# speedup vs baseline: 4.6804x; 4.6804x over previous
import functools

import jax
import jax.numpy as jnp
from jax import lax
from jax.experimental import pallas as pl
from jax.experimental.pallas import tpu as pltpu
from jax.experimental.pallas import tpu_sc as plsc

N = 10000
E = 320000
D_IN = 128
D_EDGE = 16
H = 64
OUT = 64
G = 16

NC, NS = 2, 16
NW = NC * NS
CH = 256
SB = 128
NSB = CH // SB
NCHUNKS = E // CH
CH_FULL = NCHUNKS // NW
CH_EXTRA = NCHUNKS - CH_FULL * NW
ROWS_PS = 624
ZR = 16
EB = 8000


def _mm(a, b):
    return lax.dot_general(a, b, (((1,), (0,)), ((), ())),
                           preferred_element_type=jnp.float32)


def _node0_body(x_ref, wn_ref, bn_ref, wa_ref, wb_ref, bm_ref,
                h_ref, p_ref, q_ref):
    h = jnp.maximum(_mm(x_ref[...], wn_ref[...]) + bn_ref[...], 0.0)
    h_ref[...] = h
    p_ref[...] = _mm(h, wa_ref[...]) + bm_ref[...]
    q_ref[...] = _mm(h, wb_ref[...])


def _edge0_body(ea_ref, we_ref, be_ref, wc_ref, t_ref):
    e0 = jnp.maximum(_mm(ea_ref[...], we_ref[...]) + be_ref[...], 0.0)
    t_ref[...] = _mm(e0, wc_ref[...])


def _upd_body(h_ref, agg_ref, wu1_ref, wu2_ref, bu_ref,
              wa_ref, wb_ref, bm_ref, h2_ref, p_ref, q_ref):
    agg = agg_ref[0] + agg_ref[1]
    h2 = jnp.maximum(_mm(h_ref[...], wu1_ref[...])
                     + _mm(agg, wu2_ref[...]) + bu_ref[...], 0.0)
    h2_ref[...] = h2
    p_ref[...] = _mm(h2, wa_ref[...]) + bm_ref[...]
    q_ref[...] = _mm(h2, wb_ref[...])


def _t_body(e_ref, wc_ref, t_ref):
    t_ref[...] = _mm(e_ref[...], wc_ref[...])


def _readout_body(pp_ref, cc_ref, w1_ref, b1_ref, w2_ref, b2_ref, o_ref):
    pooled_sum = pp_ref[0] + pp_ref[1]
    counts = cc_ref[0] + cc_ref[1]
    pooled = pooled_sum / jnp.maximum(counts, 1.0)
    hh = jnp.maximum(_mm(pooled, w1_ref[...]) + b1_ref[...], 0.0)
    o_ref[...] = _mm(hh, w2_ref[...]) + b2_ref[...]


_f32 = jnp.float32


def _sds(shape):
    return jax.ShapeDtypeStruct(shape, _f32)


_node0 = pl.pallas_call(
    _node0_body,
    out_shape=(_sds((N, H)), _sds((N, H)), _sds((N, H))))

_edge0 = pl.pallas_call(
    _edge0_body,
    grid=(E // EB,),
    in_specs=[
        pl.BlockSpec((EB, D_EDGE), lambda i: (i, 0)),
        pl.BlockSpec((D_EDGE, H), lambda i: (0, 0)),
        pl.BlockSpec((1, H), lambda i: (0, 0)),
        pl.BlockSpec((H, H), lambda i: (0, 0)),
    ],
    out_specs=pl.BlockSpec((EB, H), lambda i: (i, 0)),
    out_shape=_sds((E, H)))

_upd = pl.pallas_call(
    _upd_body,
    out_shape=(_sds((N, H)), _sds((N, H)), _sds((N, H))))

_tmat = pl.pallas_call(
    _t_body,
    grid=(E // EB,),
    in_specs=[
        pl.BlockSpec((EB, H), lambda i: (i, 0)),
        pl.BlockSpec((H, H), lambda i: (0, 0)),
    ],
    out_specs=pl.BlockSpec((EB, H), lambda i: (i, 0)),
    out_shape=_sds((E, H)))

_readout = pl.pallas_call(
    _readout_body,
    out_shape=_sds((G, OUT)))


_MESH = plsc.VectorSubcoreMesh(core_axis_name="c", subcore_axis_name="s",
                               num_cores=NC, num_subcores=NS)


def _zero_rows(ref, rows):
    def body(r, _):
        for jj in range(H // 16):
            ref[r, pl.ds(jj * 16, 16)] = jnp.zeros((16,), _f32)
        return 0
    lax.fori_loop(0, rows, body, 0, unroll=True if rows <= 2 else False)


def _sc_layer_body(t_hbm, p_hbm, q_hbm, src_hbm, dst_hbm,
                   e_hbm, agg_hbm,
                   t_v, gp_v, gq_v, sidx_v, didx_v, zero_v, agg_sh, sem):
    cid = lax.axis_index("c")
    sid = lax.axis_index("s")
    wid = sid * NC + cid

    _zero_rows(zero_v, ZR)
    nz = ROWS_PS // ZR + jnp.where(sid == NS - 1, 1, 0)

    def zcp(m, _):
        pltpu.sync_copy(zero_v,
                        agg_sh.at[pl.ds(sid * ROWS_PS + m * ZR, ZR)])
        return 0
    lax.fori_loop(0, nz, zcp, 0)
    plsc.subcore_barrier()

    nch = CH_FULL + jnp.where(wid < CH_EXTRA, 1, 0)

    def chunk(i, _):
        c = wid + i * NW
        base = c * CH
        pltpu.sync_copy(src_hbm.at[c], sidx_v)
        pltpu.sync_copy(dst_hbm.at[c], didx_v)
        cps = [pltpu.async_copy(t_hbm.at[pl.ds(base, CH)], t_v, sem)]
        for j in range(NSB):
            cps.append(pltpu.async_copy(
                p_hbm.at[sidx_v.at[j]], gp_v.at[pl.ds(j * SB, SB)], sem))
            cps.append(pltpu.async_copy(
                q_hbm.at[didx_v.at[j]], gq_v.at[pl.ds(j * SB, SB)], sem))
        for cp in cps:
            cp.wait()

        def rows(r, _):
            for jj in range(H // 16):
                s = pl.ds(jj * 16, 16)
                t_v[r, s] = jnp.maximum(
                    t_v[r, s] + gp_v[r, s] + gq_v[r, s], 0.0)
            return 0
        lax.fori_loop(0, CH, rows, 0)

        pltpu.sync_copy(t_v, e_hbm.at[pl.ds(base, CH)])
        for j in range(NSB):
            pltpu.sync_copy(t_v.at[pl.ds(j * SB, SB)],
                            agg_sh.at[didx_v.at[j]], add=True)
        return 0

    lax.fori_loop(0, nch, chunk, 0)
    plsc.subcore_barrier()

    @pl.when(sid < NS - 1)
    def _():
        pltpu.sync_copy(agg_sh.at[pl.ds(sid * ROWS_PS, ROWS_PS)],
                        agg_hbm.at[cid].at[pl.ds(sid * ROWS_PS, ROWS_PS)])

    @pl.when(sid == NS - 1)
    def _():
        pltpu.sync_copy(
            agg_sh.at[pl.ds((NS - 1) * ROWS_PS, N - (NS - 1) * ROWS_PS)],
            agg_hbm.at[cid].at[pl.ds((NS - 1) * ROWS_PS,
                                     N - (NS - 1) * ROWS_PS)])


_sc_layer = pl.kernel(
    _sc_layer_body,
    out_type=(jax.ShapeDtypeStruct((E, H), _f32),
              jax.ShapeDtypeStruct((NC, N, H), _f32)),
    mesh=_MESH,
    compiler_params=pltpu.CompilerParams(use_tc_tiling_on_sc=False, needs_layout_passes=False),
    scratch_types=[
        pltpu.VMEM((CH, H), _f32),
        pltpu.VMEM((CH, H), _f32),
        pltpu.VMEM((CH, H), _f32),
        pltpu.VMEM((NSB, SB), jnp.int32),
        pltpu.VMEM((NSB, SB), jnp.int32),
        pltpu.VMEM((ZR, H), _f32),
        pltpu.VMEM_SHARED((N, H), _f32),
        pltpu.SemaphoreType.DMA,
    ])


def _sc_final_body(t_hbm, p_hbm, q_hbm, src_hbm, dst_hbm, batch_hbm,
                   pool_hbm, cnt_hbm,
                   t_v, gp_v, gq_v, sidx_v, didx_v, gidx_v, batch_v, ones_v,
                   pool_sh, cnt_sh, sem):
    cid = lax.axis_index("c")
    sid = lax.axis_index("s")
    wid = sid * NC + cid

    pltpu.sync_copy(batch_hbm, batch_v)

    def ones_rows(r, _):
        for jj in range(H // 16):
            ones_v[r, pl.ds(jj * 16, 16)] = jnp.ones((16,), _f32)
        return 0
    lax.fori_loop(0, SB, ones_rows, 0)
    _zero_rows(gp_v, 1)
    pltpu.sync_copy(gp_v.at[pl.ds(0, 1)], pool_sh.at[pl.ds(sid, 1)])
    pltpu.sync_copy(gp_v.at[pl.ds(0, 1)], cnt_sh.at[pl.ds(sid, 1)])
    plsc.subcore_barrier()

    nch = CH_FULL + jnp.where(wid < CH_EXTRA, 1, 0)

    def chunk(i, _):
        c = wid + i * NW
        base = c * CH
        pltpu.sync_copy(src_hbm.at[c], sidx_v)
        pltpu.sync_copy(dst_hbm.at[c], didx_v)
        cps = [pltpu.async_copy(t_hbm.at[pl.ds(base, CH)], t_v, sem)]
        for j in range(NSB):
            cps.append(pltpu.async_copy(
                p_hbm.at[sidx_v.at[j]], gp_v.at[pl.ds(j * SB, SB)], sem))
            cps.append(pltpu.async_copy(
                q_hbm.at[didx_v.at[j]], gq_v.at[pl.ds(j * SB, SB)], sem))
        for j in range(NSB):
            for m in range(SB // 16):
                s = pl.ds(m * 16, 16)
                gidx_v[j, s] = plsc.load_gather(batch_v, [sidx_v[j, s]])
        for cp in cps:
            cp.wait()

        def rows(r, _):
            for jj in range(H // 16):
                s = pl.ds(jj * 16, 16)
                t_v[r, s] = jnp.maximum(
                    t_v[r, s] + gp_v[r, s] + gq_v[r, s], 0.0)
            return 0
        lax.fori_loop(0, CH, rows, 0)

        for j in range(NSB):
            pltpu.sync_copy(t_v.at[pl.ds(j * SB, SB)],
                            pool_sh.at[gidx_v.at[j]], add=True)
            pltpu.sync_copy(ones_v,
                            cnt_sh.at[gidx_v.at[j]], add=True)
        return 0

    lax.fori_loop(0, nch, chunk, 0)
    plsc.subcore_barrier()

    @pl.when(sid == 0)
    def _():
        pltpu.sync_copy(pool_sh, pool_hbm.at[cid])
        pltpu.sync_copy(cnt_sh, cnt_hbm.at[cid])


_sc_final = pl.kernel(
    _sc_final_body,
    out_type=(jax.ShapeDtypeStruct((NC, G, H), _f32),
              jax.ShapeDtypeStruct((NC, G, H), _f32)),
    mesh=_MESH,
    compiler_params=pltpu.CompilerParams(use_tc_tiling_on_sc=False, needs_layout_passes=False),
    scratch_types=[
        pltpu.VMEM((CH, H), _f32),
        pltpu.VMEM((CH, H), _f32),
        pltpu.VMEM((CH, H), _f32),
        pltpu.VMEM((NSB, SB), jnp.int32),
        pltpu.VMEM((NSB, SB), jnp.int32),
        pltpu.VMEM((NSB, SB), jnp.int32),
        pltpu.VMEM((N,), jnp.int32),
        pltpu.VMEM((SB, H), _f32),
        pltpu.VMEM_SHARED((G, H), _f32),
        pltpu.VMEM_SHARED((G, H), _f32),
        pltpu.SemaphoreType.DMA,
    ])


def kernel(x, edge_attr, params, edge_index, batch):
    src2 = edge_index[0].reshape(NCHUNKS, NSB, SB)
    dst2 = edge_index[1].reshape(NCHUNKS, NSB, SB)

    def msplit(l):
        w = params[f'W_msg_{l}']
        return w[:H], w[H:2 * H], w[2 * H:]

    def usplit(l):
        w = params[f'W_upd_{l}']
        return w[:H], w[H:]

    def b2d(b):
        return b.reshape(1, H)

    wa0, wb0, wc0 = msplit(0)
    h, p, q = _node0(x, params['Wn_enc'], b2d(params['bn_enc']),
                     wa0, wb0, b2d(params['b_msg_0']))
    t = _edge0(edge_attr, params['We_enc'], b2d(params['be_enc']), wc0)

    for l in range(2):
        e, aggp = _sc_layer(t, p, q, src2, dst2)
        wu1, wu2 = usplit(l)
        wa, wb, wc = msplit(l + 1)
        h, p, q = _upd(h, aggp, wu1, wu2, b2d(params[f'b_upd_{l}']),
                       wa, wb, b2d(params[f'b_msg_{l + 1}']))
        t = _tmat(e, wc)

    poolp, cntp = _sc_final(t, p, q, src2, dst2, batch)
    out = _readout(poolp, cntp, params['W_r1'], b2d(params['b_r1']),
                   params['W_r2'], b2d(params['b_r2']))
    return out


# speedup vs baseline: 5.1109x; 1.0920x over previous
import jax
import jax.numpy as jnp
from jax import lax
from jax.experimental import pallas as pl
from jax.experimental.pallas import tpu as pltpu
from jax.experimental.pallas import tpu_sc as plsc

N = 10000
E = 320000
D_IN = 128
D_EDGE = 16
H = 64
OUT = 64
G = 16

NC, NS = 2, 16
NW = NC * NS
CH = 256
CHH = CH // 2
SB = 128
NCHUNKS = E // CH
CH_FULL = NCHUNKS // NW
CH_EXTRA = NCHUNKS - CH_FULL * NW
ROWS_PS = 624
ZR = 16
EB = 8000
EH = E // 2

_f32 = jnp.float32


def _mm(a, b):
    return lax.dot_general(a, b, (((1,), (0,)), ((), ())),
                           preferred_element_type=jnp.float32)


def _node0_body(x_ref, wn_ref, bn_ref, wa_ref, wb_ref, bm_ref,
                h_ref, p_ref, q_ref):
    h = jnp.maximum(_mm(x_ref[...], wn_ref[...]) + bn_ref[...], 0.0)
    h_ref[...] = h
    p_ref[...] = _mm(h, wa_ref[...]) + bm_ref[...]
    q_ref[...] = _mm(h, wb_ref[...])


def _edge0_body(lo_ref, hi_ref, we_ref, be_ref, wc_ref, t_ref):
    lo = jnp.maximum(_mm(lo_ref[...], we_ref[...]) + be_ref[...], 0.0)
    hi = jnp.maximum(_mm(hi_ref[...], we_ref[...]) + be_ref[...], 0.0)
    t_ref[...] = jnp.concatenate(
        [_mm(lo, wc_ref[...]), _mm(hi, wc_ref[...])], axis=1)


def _upd_body(h_ref, agg_ref, wu1_ref, wu2_ref, bu_ref,
              wa_ref, wb_ref, bm_ref, h2_ref, p_ref, q_ref):
    agg = agg_ref[0] + agg_ref[1]
    h2 = jnp.maximum(_mm(h_ref[...], wu1_ref[...])
                     + _mm(agg, wu2_ref[...]) + bu_ref[...], 0.0)
    h2_ref[...] = h2
    p_ref[...] = _mm(h2, wa_ref[...]) + bm_ref[...]
    q_ref[...] = _mm(h2, wb_ref[...])


def _t_body(e_ref, wc2_ref, t_ref):
    t_ref[...] = _mm(e_ref[...], wc2_ref[...])


def _readout_body(pp_ref, cc_ref, w1_ref, b1_ref, w2_ref, b2_ref, o_ref):
    pooled_sum = pp_ref[0] + pp_ref[1]
    counts = cc_ref[0] + cc_ref[1]
    pooled = pooled_sum / jnp.maximum(counts, 1.0)
    hh = jnp.maximum(_mm(pooled, w1_ref[...]) + b1_ref[...], 0.0)
    o_ref[...] = _mm(hh, w2_ref[...]) + b2_ref[...]


def _sds(shape):
    return jax.ShapeDtypeStruct(shape, _f32)


_node0 = pl.pallas_call(
    _node0_body,
    out_shape=(_sds((N, H)), _sds((N, H)), _sds((N, H))))

_EBH = EB // 2

_edge0 = pl.pallas_call(
    _edge0_body,
    grid=(EH // _EBH,),
    in_specs=[
        pl.BlockSpec((_EBH, D_EDGE), lambda i: (i, 0)),
        pl.BlockSpec((_EBH, D_EDGE), lambda i: (i + EH // _EBH, 0)),
        pl.BlockSpec((D_EDGE, H), lambda i: (0, 0)),
        pl.BlockSpec((1, H), lambda i: (0, 0)),
        pl.BlockSpec((H, H), lambda i: (0, 0)),
    ],
    out_specs=pl.BlockSpec((_EBH, 128), lambda i: (i, 0)),
    out_shape=_sds((EH, 128)))

_upd = pl.pallas_call(
    _upd_body,
    out_shape=(_sds((N, H)), _sds((N, H)), _sds((N, H))))

_tmat = pl.pallas_call(
    _t_body,
    grid=(E // EB,),
    in_specs=[
        pl.BlockSpec((EB // 2, 128), lambda i: (i, 0)),
        pl.BlockSpec((128, 128), lambda i: (0, 0)),
    ],
    out_specs=pl.BlockSpec((EB // 2, 128), lambda i: (i, 0)),
    out_shape=_sds((EH, 128)))

_readout = pl.pallas_call(
    _readout_body,
    out_shape=_sds((G, OUT)))


_MESH = plsc.VectorSubcoreMesh(core_axis_name="c", subcore_axis_name="s",
                               num_cores=NC, num_subcores=NS)

_SC_PARAMS = pltpu.CompilerParams(use_tc_tiling_on_sc=False,
                                  needs_layout_passes=False)


def _zero_rows(ref, rows):
    def body(r, _):
        for jj in range(H // 16):
            ref[r, pl.ds(jj * 16, 16)] = jnp.zeros((16,), _f32)
        return 0
    lax.fori_loop(0, rows, body, 0)


def _sc_layer_body(t_hbm, p_hbm, q_hbm, idx_hbm,
                   e_hbm, agg_hbm,
                   t_v, gp_v, gq_v, e_v, idx_v, zero_v, agg_sh, sem):
    cid = lax.axis_index("c")
    sid = lax.axis_index("s")
    wid = sid * NC + cid

    _zero_rows(zero_v, ZR)
    nz = ROWS_PS // ZR + jnp.where(sid == NS - 1, 1, 0)

    def zcp(m, _):
        pltpu.sync_copy(zero_v,
                        agg_sh.at[pl.ds(sid * ROWS_PS + m * ZR, ZR)])
        return 0
    lax.fori_loop(0, nz, zcp, 0)
    plsc.subcore_barrier()

    nch = CH_FULL + jnp.where(wid < CH_EXTRA, 1, 0)

    def chunk(i, _):
        c = wid + i * NW
        baseh = c * CHH
        pltpu.sync_copy(idx_hbm.at[c], idx_v)
        cps = [
            pltpu.async_copy(t_hbm.at[pl.ds(baseh, CHH)], t_v, sem),
            pltpu.async_copy(p_hbm.at[idx_v.at[0]],
                             gp_v.at[pl.ds(0, SB)], sem),
            pltpu.async_copy(p_hbm.at[idx_v.at[1]],
                             gp_v.at[pl.ds(SB, SB)], sem),
            pltpu.async_copy(q_hbm.at[idx_v.at[2]],
                             gq_v.at[pl.ds(0, SB)], sem),
            pltpu.async_copy(q_hbm.at[idx_v.at[3]],
                             gq_v.at[pl.ds(SB, SB)], sem),
        ]
        for cp in cps:
            cp.wait()

        def rows(r, _):
            for half in range(2):
                for jj in range(4):
                    sv = pl.ds((half * 4 + jj) * 16, 16)
                    sg = pl.ds(jj * 16, 16)
                    v = jnp.maximum(
                        t_v[r, sv] + gp_v[half * SB + r, sg]
                        + gq_v[half * SB + r, sg], 0.0)
                    t_v[r, sv] = v
                    e_v[half * SB + r, sg] = v
            return 0
        lax.fori_loop(0, CHH, rows, 0)

        pltpu.sync_copy(t_v, e_hbm.at[pl.ds(baseh, CHH)])
        pltpu.sync_copy(e_v.at[pl.ds(0, SB)],
                        agg_sh.at[idx_v.at[2]], add=True)
        pltpu.sync_copy(e_v.at[pl.ds(SB, SB)],
                        agg_sh.at[idx_v.at[3]], add=True)
        return 0

    lax.fori_loop(0, nch, chunk, 0)
    plsc.subcore_barrier()

    @pl.when(sid < NS - 1)
    def _():
        pltpu.sync_copy(agg_sh.at[pl.ds(sid * ROWS_PS, ROWS_PS)],
                        agg_hbm.at[cid].at[pl.ds(sid * ROWS_PS, ROWS_PS)])

    @pl.when(sid == NS - 1)
    def _():
        pltpu.sync_copy(
            agg_sh.at[pl.ds((NS - 1) * ROWS_PS, N - (NS - 1) * ROWS_PS)],
            agg_hbm.at[cid].at[pl.ds((NS - 1) * ROWS_PS,
                                     N - (NS - 1) * ROWS_PS)])


_sc_layer = pl.kernel(
    _sc_layer_body,
    out_type=(jax.ShapeDtypeStruct((EH, 128), _f32),
              jax.ShapeDtypeStruct((NC, N, H), _f32)),
    mesh=_MESH,
    compiler_params=_SC_PARAMS,
    scratch_types=[
        pltpu.VMEM((CHH, 128), _f32),
        pltpu.VMEM((CH, H), _f32),
        pltpu.VMEM((CH, H), _f32),
        pltpu.VMEM((CH, H), _f32),
        pltpu.VMEM((4, SB), jnp.int32),
        pltpu.VMEM((ZR, H), _f32),
        pltpu.VMEM_SHARED((N, H), _f32),
        pltpu.SemaphoreType.DMA,
    ])


def _sc_final_body(t_hbm, p_hbm, q_hbm, idx_hbm, batch_hbm,
                   pool_hbm, cnt_hbm,
                   t_v, gp_v, gq_v, e_v, idx_v, gidx_v, batch_v, ones_v,
                   pool_sh, cnt_sh, sem):
    cid = lax.axis_index("c")
    sid = lax.axis_index("s")
    wid = sid * NC + cid

    pltpu.sync_copy(batch_hbm, batch_v)

    def ones_rows(r, _):
        for jj in range(H // 16):
            ones_v[r, pl.ds(jj * 16, 16)] = jnp.ones((16,), _f32)
        return 0
    lax.fori_loop(0, SB, ones_rows, 0)
    _zero_rows(gp_v, 1)
    pltpu.sync_copy(gp_v.at[pl.ds(0, 1)], pool_sh.at[pl.ds(sid, 1)])
    pltpu.sync_copy(gp_v.at[pl.ds(0, 1)], cnt_sh.at[pl.ds(sid, 1)])
    plsc.subcore_barrier()

    nch = CH_FULL + jnp.where(wid < CH_EXTRA, 1, 0)

    def chunk(i, _):
        c = wid + i * NW
        baseh = c * CHH
        pltpu.sync_copy(idx_hbm.at[c], idx_v)
        cps = [
            pltpu.async_copy(t_hbm.at[pl.ds(baseh, CHH)], t_v, sem),
            pltpu.async_copy(p_hbm.at[idx_v.at[0]],
                             gp_v.at[pl.ds(0, SB)], sem),
            pltpu.async_copy(p_hbm.at[idx_v.at[1]],
                             gp_v.at[pl.ds(SB, SB)], sem),
            pltpu.async_copy(q_hbm.at[idx_v.at[2]],
                             gq_v.at[pl.ds(0, SB)], sem),
            pltpu.async_copy(q_hbm.at[idx_v.at[3]],
                             gq_v.at[pl.ds(SB, SB)], sem),
        ]
        for half in range(2):
            for m in range(SB // 16):
                s = pl.ds(m * 16, 16)
                gidx_v[half, s] = plsc.load_gather(batch_v, [idx_v[half, s]])
        for cp in cps:
            cp.wait()

        def rows(r, _):
            for half in range(2):
                for jj in range(4):
                    sv = pl.ds((half * 4 + jj) * 16, 16)
                    sg = pl.ds(jj * 16, 16)
                    e_v[half * SB + r, sg] = jnp.maximum(
                        t_v[r, sv] + gp_v[half * SB + r, sg]
                        + gq_v[half * SB + r, sg], 0.0)
            return 0
        lax.fori_loop(0, CHH, rows, 0)

        pltpu.sync_copy(e_v.at[pl.ds(0, SB)],
                        pool_sh.at[gidx_v.at[0]], add=True)
        pltpu.sync_copy(e_v.at[pl.ds(SB, SB)],
                        pool_sh.at[gidx_v.at[1]], add=True)
        pltpu.sync_copy(ones_v, cnt_sh.at[gidx_v.at[0]], add=True)
        pltpu.sync_copy(ones_v, cnt_sh.at[gidx_v.at[1]], add=True)
        return 0

    lax.fori_loop(0, nch, chunk, 0)
    plsc.subcore_barrier()

    @pl.when(sid == 0)
    def _():
        pltpu.sync_copy(pool_sh, pool_hbm.at[cid])
        pltpu.sync_copy(cnt_sh, cnt_hbm.at[cid])


_sc_final = pl.kernel(
    _sc_final_body,
    out_type=(jax.ShapeDtypeStruct((NC, G, H), _f32),
              jax.ShapeDtypeStruct((NC, G, H), _f32)),
    mesh=_MESH,
    compiler_params=_SC_PARAMS,
    scratch_types=[
        pltpu.VMEM((CHH, 128), _f32),
        pltpu.VMEM((CH, H), _f32),
        pltpu.VMEM((CH, H), _f32),
        pltpu.VMEM((CH, H), _f32),
        pltpu.VMEM((4, SB), jnp.int32),
        pltpu.VMEM((2, SB), jnp.int32),
        pltpu.VMEM((N,), jnp.int32),
        pltpu.VMEM((SB, H), _f32),
        pltpu.VMEM_SHARED((G, H), _f32),
        pltpu.VMEM_SHARED((G, H), _f32),
        pltpu.SemaphoreType.DMA,
    ])


def kernel(x, edge_attr, params, edge_index, batch):
    src, dst = edge_index[0], edge_index[1]
    idx = jnp.stack([src[:EH].reshape(NCHUNKS, SB),
                     src[EH:].reshape(NCHUNKS, SB),
                     dst[:EH].reshape(NCHUNKS, SB),
                     dst[EH:].reshape(NCHUNKS, SB)], axis=1)

    def msplit(l):
        w = params[f'W_msg_{l}']
        return w[:H], w[H:2 * H], w[2 * H:]

    def usplit(l):
        w = params[f'W_upd_{l}']
        return w[:H], w[H:]

    def b2d(b):
        return b.reshape(1, H)

    def blockdiag2(w):
        z = jnp.zeros((H, H), _f32)
        return jnp.concatenate([jnp.concatenate([w, z], axis=1),
                                jnp.concatenate([z, w], axis=1)], axis=0)

    wa0, wb0, wc0 = msplit(0)
    h, p, q = _node0(x, params['Wn_enc'], b2d(params['bn_enc']),
                     wa0, wb0, b2d(params['b_msg_0']))
    t = _edge0(edge_attr, edge_attr, params['We_enc'],
               b2d(params['be_enc']), wc0)

    for l in range(2):
        e, aggp = _sc_layer(t, p, q, idx)
        wu1, wu2 = usplit(l)
        wa, wb, wc = msplit(l + 1)
        h, p, q = _upd(h, aggp, wu1, wu2, b2d(params[f'b_upd_{l}']),
                       wa, wb, b2d(params[f'b_msg_{l + 1}']))
        t = _tmat(e, blockdiag2(wc))

    poolp, cntp = _sc_final(t, p, q, idx, batch)
    out = _readout(poolp, cntp, params['W_r1'], b2d(params['b_r1']),
                   params['W_r2'], b2d(params['b_r2']))
    return out


# speedup vs baseline: 8.4348x; 1.6503x over previous
import jax
import jax.numpy as jnp
from jax import lax
from jax.experimental import pallas as pl
from jax.experimental.pallas import tpu as pltpu
from jax.experimental.pallas import tpu_sc as plsc

N = 10000
E = 320000
D_IN = 128
D_EDGE = 16
H = 64
OUT = 64
G = 16

NC, NS = 2, 16
NW = NC * NS
CH = 256
CHH = CH // 2
SB = 128
NCHUNKS = E // CH
CH_FULL = NCHUNKS // NW
CH_EXTRA = NCHUNKS - CH_FULL * NW
ROWS_PS = 624
ZR = 16
EB = 8000
EH = E // 2

_f32 = jnp.float32


def _mm(a, b):
    return lax.dot_general(a, b, (((1,), (0,)), ((), ())),
                           preferred_element_type=jnp.float32)


def _node0_body(x_ref, wn_ref, bn_ref, wa_ref, wb_ref, bm_ref,
                h_ref, p_ref, q_ref):
    h = jnp.maximum(_mm(x_ref[...], wn_ref[...]) + bn_ref[...], 0.0)
    h_ref[...] = h
    p_ref[...] = _mm(h, wa_ref[...]) + bm_ref[...]
    q_ref[...] = _mm(h, wb_ref[...])


def _dotT(a, b):
    return lax.dot_general(a, b, (((0,), (0,)), ((), ())),
                           preferred_element_type=jnp.float32)


def _edge0_body(lo_ref, hi_ref, we_ref, be_ref, wc_ref, t_ref):
    lo = jnp.maximum(_dotT(lo_ref[...], we_ref[...]) + be_ref[...], 0.0)
    hi = jnp.maximum(_dotT(hi_ref[...], we_ref[...]) + be_ref[...], 0.0)
    t_ref[...] = jnp.concatenate(
        [_mm(lo, wc_ref[...]), _mm(hi, wc_ref[...])], axis=1)


def _upd_body(h_ref, agg_ref, wu1_ref, wu2_ref, bu_ref,
              wa_ref, wb_ref, bm_ref, h2_ref, p_ref, q_ref):
    agg = agg_ref[0] + agg_ref[1]
    h2 = jnp.maximum(_mm(h_ref[...], wu1_ref[...])
                     + _mm(agg, wu2_ref[...]) + bu_ref[...], 0.0)
    h2_ref[...] = h2
    p_ref[...] = _mm(h2, wa_ref[...]) + bm_ref[...]
    q_ref[...] = _mm(h2, wb_ref[...])


def _t_body(e_ref, wc2_ref, t_ref):
    t_ref[...] = _mm(e_ref[...], wc2_ref[...])


def _readout_body(pp_ref, cc_ref, w1_ref, b1_ref, w2_ref, b2_ref, o_ref):
    pooled_sum = pp_ref[0] + pp_ref[1]
    counts = cc_ref[0] + cc_ref[1]
    pooled = pooled_sum / jnp.maximum(counts, 1.0)
    hh = jnp.maximum(_mm(pooled, w1_ref[...]) + b1_ref[...], 0.0)
    o_ref[...] = _mm(hh, w2_ref[...]) + b2_ref[...]


def _sds(shape):
    return jax.ShapeDtypeStruct(shape, _f32)


_node0 = pl.pallas_call(
    _node0_body,
    out_shape=(_sds((N, H)), _sds((N, H)), _sds((N, H))))

_EBH = 16000

_edge0 = pl.pallas_call(
    _edge0_body,
    grid=(EH // _EBH,),
    in_specs=[
        pl.BlockSpec((D_EDGE, _EBH), lambda i: (0, i)),
        pl.BlockSpec((D_EDGE, _EBH), lambda i: (0, i + EH // _EBH)),
        pl.BlockSpec((D_EDGE, H), lambda i: (0, 0)),
        pl.BlockSpec((1, H), lambda i: (0, 0)),
        pl.BlockSpec((H, H), lambda i: (0, 0)),
    ],
    out_specs=pl.BlockSpec((_EBH, 128), lambda i: (i, 0)),
    out_shape=_sds((EH, 128)))

_upd = pl.pallas_call(
    _upd_body,
    out_shape=(_sds((N, H)), _sds((N, H)), _sds((N, H))))

_tmat = pl.pallas_call(
    _t_body,
    grid=(E // EB,),
    in_specs=[
        pl.BlockSpec((EB // 2, 128), lambda i: (i, 0)),
        pl.BlockSpec((128, 128), lambda i: (0, 0)),
    ],
    out_specs=pl.BlockSpec((EB // 2, 128), lambda i: (i, 0)),
    out_shape=_sds((EH, 128)))

_readout = pl.pallas_call(
    _readout_body,
    out_shape=_sds((G, OUT)))


_MESH = plsc.VectorSubcoreMesh(core_axis_name="c", subcore_axis_name="s",
                               num_cores=NC, num_subcores=NS)

_SC_PARAMS = pltpu.CompilerParams(use_tc_tiling_on_sc=False,
                                  needs_layout_passes=False)


def _zero_rows(ref, rows):
    def body(r, _):
        for jj in range(H // 16):
            ref[r, pl.ds(jj * 16, 16)] = jnp.zeros((16,), _f32)
        return 0
    lax.fori_loop(0, rows, body, 0)


def _sc_layer_body(t_hbm, p_hbm, q_hbm, idx_hbm,
                   e_hbm, agg_hbm,
                   t_v, gp_v, gq_v, e_v, idx_v, zero_v, agg_sh, sem):
    cid = lax.axis_index("c")
    sid = lax.axis_index("s")
    wid = sid * NC + cid

    _zero_rows(zero_v, ZR)
    nz = ROWS_PS // ZR + jnp.where(sid == NS - 1, 1, 0)

    def zcp(m, _):
        pltpu.sync_copy(zero_v,
                        agg_sh.at[pl.ds(sid * ROWS_PS + m * ZR, ZR)])
        return 0
    lax.fori_loop(0, nz, zcp, 0)
    plsc.subcore_barrier()

    nch = CH_FULL + jnp.where(wid < CH_EXTRA, 1, 0)

    def chunk(i, _):
        c = wid + i * NW
        baseh = c * CHH
        pltpu.sync_copy(idx_hbm.at[c], idx_v)
        cps = [
            pltpu.async_copy(t_hbm.at[pl.ds(baseh, CHH)], t_v, sem),
            pltpu.async_copy(p_hbm.at[idx_v.at[0]],
                             gp_v.at[pl.ds(0, SB)], sem),
            pltpu.async_copy(p_hbm.at[idx_v.at[1]],
                             gp_v.at[pl.ds(SB, SB)], sem),
            pltpu.async_copy(q_hbm.at[idx_v.at[2]],
                             gq_v.at[pl.ds(0, SB)], sem),
            pltpu.async_copy(q_hbm.at[idx_v.at[3]],
                             gq_v.at[pl.ds(SB, SB)], sem),
        ]
        for cp in cps:
            cp.wait()

        @plsc.parallel_loop(0, CHH, step=1, unroll=2)
        def rows(r):
            for half in range(2):
                for jj in range(4):
                    sv = pl.ds((half * 4 + jj) * 16, 16)
                    sg = pl.ds(jj * 16, 16)
                    v = jnp.maximum(
                        t_v[r, sv] + gp_v[half * SB + r, sg]
                        + gq_v[half * SB + r, sg], 0.0)
                    t_v[r, sv] = v
                    e_v[half * SB + r, sg] = v

        pltpu.sync_copy(t_v, e_hbm.at[pl.ds(baseh, CHH)])
        pltpu.sync_copy(e_v.at[pl.ds(0, SB)],
                        agg_sh.at[idx_v.at[2]], add=True)
        pltpu.sync_copy(e_v.at[pl.ds(SB, SB)],
                        agg_sh.at[idx_v.at[3]], add=True)
        return 0

    lax.fori_loop(0, nch, chunk, 0)
    plsc.subcore_barrier()

    @pl.when(sid < NS - 1)
    def _():
        pltpu.sync_copy(agg_sh.at[pl.ds(sid * ROWS_PS, ROWS_PS)],
                        agg_hbm.at[cid].at[pl.ds(sid * ROWS_PS, ROWS_PS)])

    @pl.when(sid == NS - 1)
    def _():
        pltpu.sync_copy(
            agg_sh.at[pl.ds((NS - 1) * ROWS_PS, N - (NS - 1) * ROWS_PS)],
            agg_hbm.at[cid].at[pl.ds((NS - 1) * ROWS_PS,
                                     N - (NS - 1) * ROWS_PS)])


_sc_layer = pl.kernel(
    _sc_layer_body,
    out_type=(jax.ShapeDtypeStruct((EH, 128), _f32),
              jax.ShapeDtypeStruct((NC, N, H), _f32)),
    mesh=_MESH,
    compiler_params=_SC_PARAMS,
    scratch_types=[
        pltpu.VMEM((CHH, 128), _f32),
        pltpu.VMEM((CH, H), _f32),
        pltpu.VMEM((CH, H), _f32),
        pltpu.VMEM((CH, H), _f32),
        pltpu.VMEM((4, SB), jnp.int32),
        pltpu.VMEM((ZR, H), _f32),
        pltpu.VMEM_SHARED((N, H), _f32),
        pltpu.SemaphoreType.DMA,
    ])


def _sc_final_body(t_hbm, p_hbm, q_hbm, idx_hbm, batch_hbm,
                   pool_hbm, cnt_hbm,
                   t_v, gp_v, gq_v, e_v, idx_v, gidx_v, batch_v, ones_v,
                   pool_sh, cnt_sh, sem):
    cid = lax.axis_index("c")
    sid = lax.axis_index("s")
    wid = sid * NC + cid

    pltpu.sync_copy(batch_hbm, batch_v)

    def ones_rows(r, _):
        for jj in range(H // 16):
            ones_v[r, pl.ds(jj * 16, 16)] = jnp.ones((16,), _f32)
        return 0
    lax.fori_loop(0, SB, ones_rows, 0)
    _zero_rows(gp_v, 1)
    pltpu.sync_copy(gp_v.at[pl.ds(0, 1)], pool_sh.at[pl.ds(sid, 1)])
    pltpu.sync_copy(gp_v.at[pl.ds(0, 1)], cnt_sh.at[pl.ds(sid, 1)])
    plsc.subcore_barrier()

    nch = CH_FULL + jnp.where(wid < CH_EXTRA, 1, 0)

    def chunk(i, _):
        c = wid + i * NW
        baseh = c * CHH
        pltpu.sync_copy(idx_hbm.at[c], idx_v)
        cps = [
            pltpu.async_copy(t_hbm.at[pl.ds(baseh, CHH)], t_v, sem),
            pltpu.async_copy(p_hbm.at[idx_v.at[0]],
                             gp_v.at[pl.ds(0, SB)], sem),
            pltpu.async_copy(p_hbm.at[idx_v.at[1]],
                             gp_v.at[pl.ds(SB, SB)], sem),
            pltpu.async_copy(q_hbm.at[idx_v.at[2]],
                             gq_v.at[pl.ds(0, SB)], sem),
            pltpu.async_copy(q_hbm.at[idx_v.at[3]],
                             gq_v.at[pl.ds(SB, SB)], sem),
        ]
        for half in range(2):
            for m in range(SB // 16):
                s = pl.ds(m * 16, 16)
                gidx_v[half, s] = plsc.load_gather(batch_v, [idx_v[half, s]])
        for cp in cps:
            cp.wait()

        @plsc.parallel_loop(0, CHH, step=1, unroll=2)
        def rows(r):
            for half in range(2):
                for jj in range(4):
                    sv = pl.ds((half * 4 + jj) * 16, 16)
                    sg = pl.ds(jj * 16, 16)
                    e_v[half * SB + r, sg] = jnp.maximum(
                        t_v[r, sv] + gp_v[half * SB + r, sg]
                        + gq_v[half * SB + r, sg], 0.0)

        pltpu.sync_copy(e_v.at[pl.ds(0, SB)],
                        pool_sh.at[gidx_v.at[0]], add=True)
        pltpu.sync_copy(e_v.at[pl.ds(SB, SB)],
                        pool_sh.at[gidx_v.at[1]], add=True)
        pltpu.sync_copy(ones_v, cnt_sh.at[gidx_v.at[0]], add=True)
        pltpu.sync_copy(ones_v, cnt_sh.at[gidx_v.at[1]], add=True)
        return 0

    lax.fori_loop(0, nch, chunk, 0)
    plsc.subcore_barrier()

    @pl.when(sid == 0)
    def _():
        pltpu.sync_copy(pool_sh, pool_hbm.at[cid])
        pltpu.sync_copy(cnt_sh, cnt_hbm.at[cid])


_sc_final = pl.kernel(
    _sc_final_body,
    out_type=(jax.ShapeDtypeStruct((NC, G, H), _f32),
              jax.ShapeDtypeStruct((NC, G, H), _f32)),
    mesh=_MESH,
    compiler_params=_SC_PARAMS,
    scratch_types=[
        pltpu.VMEM((CHH, 128), _f32),
        pltpu.VMEM((CH, H), _f32),
        pltpu.VMEM((CH, H), _f32),
        pltpu.VMEM((CH, H), _f32),
        pltpu.VMEM((4, SB), jnp.int32),
        pltpu.VMEM((2, SB), jnp.int32),
        pltpu.VMEM((N,), jnp.int32),
        pltpu.VMEM((SB, H), _f32),
        pltpu.VMEM_SHARED((G, H), _f32),
        pltpu.VMEM_SHARED((G, H), _f32),
        pltpu.SemaphoreType.DMA,
    ])


def kernel(x, edge_attr, params, edge_index, batch):
    src, dst = edge_index[0], edge_index[1]
    idx = jnp.stack([src[:EH].reshape(NCHUNKS, SB),
                     src[EH:].reshape(NCHUNKS, SB),
                     dst[:EH].reshape(NCHUNKS, SB),
                     dst[EH:].reshape(NCHUNKS, SB)], axis=1)

    def msplit(l):
        w = params[f'W_msg_{l}']
        return w[:H], w[H:2 * H], w[2 * H:]

    def usplit(l):
        w = params[f'W_upd_{l}']
        return w[:H], w[H:]

    def b2d(b):
        return b.reshape(1, H)

    def blockdiag2(w):
        z = jnp.zeros((H, H), _f32)
        return jnp.concatenate([jnp.concatenate([w, z], axis=1),
                                jnp.concatenate([z, w], axis=1)], axis=0)

    wa0, wb0, wc0 = msplit(0)
    h, p, q = _node0(x, params['Wn_enc'], b2d(params['bn_enc']),
                     wa0, wb0, b2d(params['b_msg_0']))
    eat = edge_attr.T
    t = _edge0(eat, eat, params['We_enc'], b2d(params['be_enc']), wc0)

    for l in range(2):
        e, aggp = _sc_layer(t, p, q, idx)
        wu1, wu2 = usplit(l)
        wa, wb, wc = msplit(l + 1)
        h, p, q = _upd(h, aggp, wu1, wu2, b2d(params[f'b_upd_{l}']),
                       wa, wb, b2d(params[f'b_msg_{l + 1}']))
        t = _tmat(e, blockdiag2(wc))

    poolp, cntp = _sc_final(t, p, q, idx, batch)
    out = _readout(poolp, cntp, params['W_r1'], b2d(params['b_r1']),
                   params['W_r2'], b2d(params['b_r2']))
    return out


# speedup vs baseline: 10.0813x; 1.1952x over previous
import jax
import jax.numpy as jnp
from jax import lax
from jax.experimental import pallas as pl
from jax.experimental.pallas import tpu as pltpu
from jax.experimental.pallas import tpu_sc as plsc

N = 10000
E = 320000
D_IN = 128
D_EDGE = 16
H = 64
OUT = 64
G = 16

NC, NS = 2, 16
NW = NC * NS
CH = 256
CHH = CH // 2
SB = 128
NCHUNKS = E // CH
CH_FULL = NCHUNKS // NW
CH_EXTRA = NCHUNKS - CH_FULL * NW
ROWS_PS = 624
ZR = 16
EB = 8000
EH = E // 2

_f32 = jnp.float32


def _mm(a, b):
    return lax.dot_general(a, b, (((1,), (0,)), ((), ())),
                           preferred_element_type=jnp.float32)


def _node0_body(x_ref, wn_ref, bn_ref, wa_ref, wb_ref, bm_ref,
                h_ref, p_ref, q_ref):
    h = jnp.maximum(_mm(x_ref[...], wn_ref[...]) + bn_ref[...], 0.0)
    h_ref[...] = h
    p_ref[...] = _mm(h, wa_ref[...]) + bm_ref[...]
    q_ref[...] = _mm(h, wb_ref[...])


def _dotT(a, b):
    return lax.dot_general(a, b, (((0,), (0,)), ((), ())),
                           preferred_element_type=jnp.float32)


def _edge0_body(lo_ref, hi_ref, we_ref, be_ref, wc_ref, t_ref):
    lo = jnp.maximum(_dotT(lo_ref[...], we_ref[...]) + be_ref[...], 0.0)
    hi = jnp.maximum(_dotT(hi_ref[...], we_ref[...]) + be_ref[...], 0.0)
    t_ref[...] = jnp.concatenate(
        [_mm(lo, wc_ref[...]), _mm(hi, wc_ref[...])], axis=1)


def _upd_body(h_ref, agg_ref, wu1_ref, wu2_ref, bu_ref,
              wa_ref, wb_ref, bm_ref, h2_ref, p_ref, q_ref):
    agg = agg_ref[0] + agg_ref[1]
    h2 = jnp.maximum(_mm(h_ref[...], wu1_ref[...])
                     + _mm(agg, wu2_ref[...]) + bu_ref[...], 0.0)
    h2_ref[...] = h2
    p_ref[...] = _mm(h2, wa_ref[...]) + bm_ref[...]
    q_ref[...] = _mm(h2, wb_ref[...])


def _t_body(e_ref, wc2_ref, t_ref):
    t_ref[...] = _mm(e_ref[...], wc2_ref[...])


def _readout_body(pp_ref, cc_ref, w1_ref, b1_ref, w2_ref, b2_ref, o_ref):
    pooled_sum = pp_ref[0] + pp_ref[1]
    counts = cc_ref[0] + cc_ref[1]
    pooled = pooled_sum / jnp.maximum(counts, 1.0)
    hh = jnp.maximum(_mm(pooled, w1_ref[...]) + b1_ref[...], 0.0)
    o_ref[...] = _mm(hh, w2_ref[...]) + b2_ref[...]


def _sds(shape):
    return jax.ShapeDtypeStruct(shape, _f32)


_node0 = pl.pallas_call(
    _node0_body,
    out_shape=(_sds((N, H)), _sds((N, H)), _sds((N, H))))

_EBH = 16000

_edge0 = pl.pallas_call(
    _edge0_body,
    grid=(EH // _EBH,),
    in_specs=[
        pl.BlockSpec((D_EDGE, _EBH), lambda i: (0, i)),
        pl.BlockSpec((D_EDGE, _EBH), lambda i: (0, i + EH // _EBH)),
        pl.BlockSpec((D_EDGE, H), lambda i: (0, 0)),
        pl.BlockSpec((1, H), lambda i: (0, 0)),
        pl.BlockSpec((H, H), lambda i: (0, 0)),
    ],
    out_specs=pl.BlockSpec((_EBH, 128), lambda i: (i, 0)),
    out_shape=_sds((EH, 128)))

_upd = pl.pallas_call(
    _upd_body,
    out_shape=(_sds((N, H)), _sds((N, H)), _sds((N, H))))

_tmat = pl.pallas_call(
    _t_body,
    grid=(E // EB,),
    in_specs=[
        pl.BlockSpec((EB // 2, 128), lambda i: (i, 0)),
        pl.BlockSpec((128, 128), lambda i: (0, 0)),
    ],
    out_specs=pl.BlockSpec((EB // 2, 128), lambda i: (i, 0)),
    out_shape=_sds((EH, 128)))

_readout = pl.pallas_call(
    _readout_body,
    out_shape=_sds((G, OUT)))


_MESH = plsc.VectorSubcoreMesh(core_axis_name="c", subcore_axis_name="s",
                               num_cores=NC, num_subcores=NS)

_SC_PARAMS = pltpu.CompilerParams(use_tc_tiling_on_sc=False,
                                  needs_layout_passes=False)


def _zero_rows(ref, rows):
    def body(r, _):
        for jj in range(H // 16):
            ref[r, pl.ds(jj * 16, 16)] = jnp.zeros((16,), _f32)
        return 0
    lax.fori_loop(0, rows, body, 0)


def _sc_layer_body(t_hbm, p_hbm, q_hbm, idx_hbm,
                   e_hbm, agg_hbm,
                   t_v, gp_a, gp_b, gq_a, gq_b, idx_a, idx_b, zero_v, agg_sh,
                   semt, seme, semga, semgb, semsa, semsb):
    cid = lax.axis_index("c")
    sid = lax.axis_index("s")
    wid = sid * NC + cid

    _zero_rows(zero_v, ZR)
    nz = ROWS_PS // ZR + jnp.where(sid == NS - 1, 1, 0)

    def zcp(m, _):
        pltpu.sync_copy(zero_v,
                        agg_sh.at[pl.ds(sid * ROWS_PS + m * ZR, ZR)])
        return 0
    lax.fori_loop(0, nz, zcp, 0)
    plsc.subcore_barrier()

    nch = CH_FULL + jnp.where(wid < CH_EXTRA, 1, 0)

    def start_gathers(idxv, gpv, gqv, sem):
        pltpu.async_copy(p_hbm.at[idxv.at[0]], gpv.at[pl.ds(0, SB)], sem)
        pltpu.async_copy(p_hbm.at[idxv.at[1]], gpv.at[pl.ds(SB, SB)], sem)
        pltpu.async_copy(q_hbm.at[idxv.at[2]], gqv.at[pl.ds(0, SB)], sem)
        pltpu.async_copy(q_hbm.at[idxv.at[3]], gqv.at[pl.ds(SB, SB)], sem)

    def drain_gathers(idxv, gpv, gqv, sem):
        pltpu.make_async_copy(p_hbm.at[idxv.at[0]],
                              gpv.at[pl.ds(0, SB)], sem).wait()
        pltpu.make_async_copy(p_hbm.at[idxv.at[1]],
                              gpv.at[pl.ds(SB, SB)], sem).wait()
        pltpu.make_async_copy(q_hbm.at[idxv.at[2]],
                              gqv.at[pl.ds(0, SB)], sem).wait()
        pltpu.make_async_copy(q_hbm.at[idxv.at[3]],
                              gqv.at[pl.ds(SB, SB)], sem).wait()

    def drain_scatters(idxv, gpv, sem):
        pltpu.make_async_copy(gpv.at[pl.ds(0, SB)],
                              agg_sh.at[idxv.at[2]], sem).wait()
        pltpu.make_async_copy(gpv.at[pl.ds(SB, SB)],
                              agg_sh.at[idxv.at[3]], sem).wait()

    pltpu.sync_copy(idx_hbm.at[wid], idx_a)
    start_gathers(idx_a, gp_a, gq_a, semga)

    sets = ((gp_a, gq_a, idx_a, semga, semsa),
            (gp_b, gq_b, idx_b, semgb, semsb))

    def chunk_body(i, gpx, gqx, idxx, semgx, semsx,
                   gpy, gqy, idxy, semgy, semsy):
        baseh = (wid + i * NW) * CHH

        @pl.when(i >= 1)
        def _():
            pltpu.make_async_copy(t_v, e_hbm.at[pl.ds(0, CHH)],
                                  seme).wait()
        tcp = pltpu.async_copy(t_hbm.at[pl.ds(baseh, CHH)], t_v, semt)

        @pl.when(i >= 1)
        def _():
            drain_scatters(idxy, gpy, semsy)

        @pl.when(i + 1 < nch)
        def _():
            pltpu.sync_copy(idx_hbm.at[wid + (i + 1) * NW], idxy)
            start_gathers(idxy, gpy, gqy, semgy)

        drain_gathers(idxx, gpx, gqx, semgx)
        tcp.wait()

        @plsc.parallel_loop(0, CHH, step=1, unroll=2)
        def rows(r):
            for half in range(2):
                for jj in range(4):
                    sv = pl.ds((half * 4 + jj) * 16, 16)
                    sg = pl.ds(jj * 16, 16)
                    v = jnp.maximum(
                        t_v[r, sv] + gpx[half * SB + r, sg]
                        + gqx[half * SB + r, sg], 0.0)
                    t_v[r, sv] = v
                    gpx[half * SB + r, sg] = v

        pltpu.async_copy(t_v, e_hbm.at[pl.ds(baseh, CHH)], seme)
        pltpu.async_copy(gpx.at[pl.ds(0, SB)],
                         agg_sh.at[idxx.at[2]], semsx, add=True)
        pltpu.async_copy(gpx.at[pl.ds(SB, SB)],
                         agg_sh.at[idxx.at[3]], semsx, add=True)

    def pair(u, _):
        for x in range(2):
            i = u * 2 + x

            @pl.when(i < nch)
            def _():
                chunk_body(i, *sets[x], *sets[1 - x])
        return 0
    lax.fori_loop(0, (CH_FULL + 2) // 2, pair, 0)

    pltpu.make_async_copy(t_v, e_hbm.at[pl.ds(0, CHH)], seme).wait()

    @pl.when(nch % 2 == 1)
    def _():
        drain_scatters(idx_a, gp_a, semsa)

    @pl.when(nch % 2 == 0)
    def _():
        drain_scatters(idx_b, gp_b, semsb)

    plsc.subcore_barrier()

    @pl.when(sid < NS - 1)
    def _():
        pltpu.sync_copy(agg_sh.at[pl.ds(sid * ROWS_PS, ROWS_PS)],
                        agg_hbm.at[cid].at[pl.ds(sid * ROWS_PS, ROWS_PS)])

    @pl.when(sid == NS - 1)
    def _():
        pltpu.sync_copy(
            agg_sh.at[pl.ds((NS - 1) * ROWS_PS, N - (NS - 1) * ROWS_PS)],
            agg_hbm.at[cid].at[pl.ds((NS - 1) * ROWS_PS,
                                     N - (NS - 1) * ROWS_PS)])


_sc_layer = pl.kernel(
    _sc_layer_body,
    out_type=(jax.ShapeDtypeStruct((EH, 128), _f32),
              jax.ShapeDtypeStruct((NC, N, H), _f32)),
    mesh=_MESH,
    compiler_params=_SC_PARAMS,
    scratch_types=[
        pltpu.VMEM((CHH, 128), _f32),
        pltpu.VMEM((CH, H), _f32),
        pltpu.VMEM((CH, H), _f32),
        pltpu.VMEM((CH, H), _f32),
        pltpu.VMEM((CH, H), _f32),
        pltpu.VMEM((4, SB), jnp.int32),
        pltpu.VMEM((4, SB), jnp.int32),
        pltpu.VMEM((ZR, H), _f32),
        pltpu.VMEM_SHARED((N, H), _f32),
        pltpu.SemaphoreType.DMA,
        pltpu.SemaphoreType.DMA,
        pltpu.SemaphoreType.DMA,
        pltpu.SemaphoreType.DMA,
        pltpu.SemaphoreType.DMA,
        pltpu.SemaphoreType.DMA,
    ])


def _sc_final_body(t_hbm, p_hbm, q_hbm, idx_hbm, batch_hbm,
                   pool_hbm, cnt_hbm,
                   t_v, gp_v, gq_v, e_v, idx_v, gidx_v, batch_v, ones_v,
                   pool_sh, cnt_sh, sem):
    cid = lax.axis_index("c")
    sid = lax.axis_index("s")
    wid = sid * NC + cid

    pltpu.sync_copy(batch_hbm, batch_v)

    def ones_rows(r, _):
        for jj in range(H // 16):
            ones_v[r, pl.ds(jj * 16, 16)] = jnp.ones((16,), _f32)
        return 0
    lax.fori_loop(0, SB, ones_rows, 0)
    _zero_rows(gp_v, 1)
    pltpu.sync_copy(gp_v.at[pl.ds(0, 1)], pool_sh.at[pl.ds(sid, 1)])
    pltpu.sync_copy(gp_v.at[pl.ds(0, 1)], cnt_sh.at[pl.ds(sid, 1)])
    plsc.subcore_barrier()

    nch = CH_FULL + jnp.where(wid < CH_EXTRA, 1, 0)

    def chunk(i, _):
        c = wid + i * NW
        baseh = c * CHH
        pltpu.sync_copy(idx_hbm.at[c], idx_v)
        cps = [
            pltpu.async_copy(t_hbm.at[pl.ds(baseh, CHH)], t_v, sem),
            pltpu.async_copy(p_hbm.at[idx_v.at[0]],
                             gp_v.at[pl.ds(0, SB)], sem),
            pltpu.async_copy(p_hbm.at[idx_v.at[1]],
                             gp_v.at[pl.ds(SB, SB)], sem),
            pltpu.async_copy(q_hbm.at[idx_v.at[2]],
                             gq_v.at[pl.ds(0, SB)], sem),
            pltpu.async_copy(q_hbm.at[idx_v.at[3]],
                             gq_v.at[pl.ds(SB, SB)], sem),
        ]
        for half in range(2):
            for m in range(SB // 16):
                s = pl.ds(m * 16, 16)
                gidx_v[half, s] = plsc.load_gather(batch_v, [idx_v[half, s]])
        for cp in cps:
            cp.wait()

        @plsc.parallel_loop(0, CHH, step=1, unroll=2)
        def rows(r):
            for half in range(2):
                for jj in range(4):
                    sv = pl.ds((half * 4 + jj) * 16, 16)
                    sg = pl.ds(jj * 16, 16)
                    e_v[half * SB + r, sg] = jnp.maximum(
                        t_v[r, sv] + gp_v[half * SB + r, sg]
                        + gq_v[half * SB + r, sg], 0.0)

        pltpu.sync_copy(e_v.at[pl.ds(0, SB)],
                        pool_sh.at[gidx_v.at[0]], add=True)
        pltpu.sync_copy(e_v.at[pl.ds(SB, SB)],
                        pool_sh.at[gidx_v.at[1]], add=True)
        pltpu.sync_copy(ones_v, cnt_sh.at[gidx_v.at[0]], add=True)
        pltpu.sync_copy(ones_v, cnt_sh.at[gidx_v.at[1]], add=True)
        return 0

    lax.fori_loop(0, nch, chunk, 0)
    plsc.subcore_barrier()

    @pl.when(sid == 0)
    def _():
        pltpu.sync_copy(pool_sh, pool_hbm.at[cid])
        pltpu.sync_copy(cnt_sh, cnt_hbm.at[cid])


_sc_final = pl.kernel(
    _sc_final_body,
    out_type=(jax.ShapeDtypeStruct((NC, G, H), _f32),
              jax.ShapeDtypeStruct((NC, G, H), _f32)),
    mesh=_MESH,
    compiler_params=_SC_PARAMS,
    scratch_types=[
        pltpu.VMEM((CHH, 128), _f32),
        pltpu.VMEM((CH, H), _f32),
        pltpu.VMEM((CH, H), _f32),
        pltpu.VMEM((CH, H), _f32),
        pltpu.VMEM((4, SB), jnp.int32),
        pltpu.VMEM((2, SB), jnp.int32),
        pltpu.VMEM((N,), jnp.int32),
        pltpu.VMEM((SB, H), _f32),
        pltpu.VMEM_SHARED((G, H), _f32),
        pltpu.VMEM_SHARED((G, H), _f32),
        pltpu.SemaphoreType.DMA,
    ])


def kernel(x, edge_attr, params, edge_index, batch):
    src, dst = edge_index[0], edge_index[1]
    idx = jnp.stack([src[:EH].reshape(NCHUNKS, SB),
                     src[EH:].reshape(NCHUNKS, SB),
                     dst[:EH].reshape(NCHUNKS, SB),
                     dst[EH:].reshape(NCHUNKS, SB)], axis=1)

    def msplit(l):
        w = params[f'W_msg_{l}']
        return w[:H], w[H:2 * H], w[2 * H:]

    def usplit(l):
        w = params[f'W_upd_{l}']
        return w[:H], w[H:]

    def b2d(b):
        return b.reshape(1, H)

    def blockdiag2(w):
        z = jnp.zeros((H, H), _f32)
        return jnp.concatenate([jnp.concatenate([w, z], axis=1),
                                jnp.concatenate([z, w], axis=1)], axis=0)

    wa0, wb0, wc0 = msplit(0)
    h, p, q = _node0(x, params['Wn_enc'], b2d(params['bn_enc']),
                     wa0, wb0, b2d(params['b_msg_0']))
    eat = edge_attr.T
    t = _edge0(eat, eat, params['We_enc'], b2d(params['be_enc']), wc0)

    for l in range(2):
        e, aggp = _sc_layer(t, p, q, idx)
        wu1, wu2 = usplit(l)
        wa, wb, wc = msplit(l + 1)
        h, p, q = _upd(h, aggp, wu1, wu2, b2d(params[f'b_upd_{l}']),
                       wa, wb, b2d(params[f'b_msg_{l + 1}']))
        t = _tmat(e, blockdiag2(wc))

    poolp, cntp = _sc_final(t, p, q, idx, batch)
    out = _readout(poolp, cntp, params['W_r1'], b2d(params['b_r1']),
                   params['W_r2'], b2d(params['b_r2']))
    return out


# speedup vs baseline: 11.6110x; 1.1517x over previous
import jax
import jax.numpy as jnp
from jax import lax
from jax.experimental import pallas as pl
from jax.experimental.pallas import tpu as pltpu
from jax.experimental.pallas import tpu_sc as plsc

N = 10000
E = 320000
D_IN = 128
D_EDGE = 16
H = 64
OUT = 64
G = 16

NC, NS = 2, 16
NW = NC * NS
CH = 256
CHH = CH // 2
SB = 128
NCHUNKS = E // CH
CH_FULL = NCHUNKS // NW
CH_EXTRA = NCHUNKS - CH_FULL * NW
ROWS_PS = 624
ZR = 16
EB = 8000
EH = E // 2

_f32 = jnp.float32


def _mm(a, b):
    return lax.dot_general(a, b, (((1,), (0,)), ((), ())),
                           preferred_element_type=jnp.float32)


def _node0_body(x_ref, wn_ref, bn_ref, wa_ref, wb_ref, bm_ref,
                h_ref, p_ref, q_ref):
    h = jnp.maximum(_mm(x_ref[...], wn_ref[...]) + bn_ref[...], 0.0)
    h_ref[...] = h
    p_ref[...] = _mm(h, wa_ref[...]) + bm_ref[...]
    q_ref[...] = _mm(h, wb_ref[...])


def _dotT(a, b):
    return lax.dot_general(a, b, (((0,), (0,)), ((), ())),
                           preferred_element_type=jnp.float32)


def _edge0_body(lo_ref, hi_ref, we_ref, be_ref, wc_ref, t_ref):
    lo = jnp.maximum(_dotT(lo_ref[...], we_ref[...]) + be_ref[...], 0.0)
    hi = jnp.maximum(_dotT(hi_ref[...], we_ref[...]) + be_ref[...], 0.0)
    t_ref[...] = jnp.concatenate(
        [_mm(lo, wc_ref[...]), _mm(hi, wc_ref[...])], axis=1)


def _upd_body(h_ref, agg_ref, wu1_ref, wu2_ref, bu_ref,
              wa_ref, wb_ref, bm_ref, h2_ref, p_ref, q_ref):
    agg = agg_ref[0] + agg_ref[1]
    h2 = jnp.maximum(_mm(h_ref[...], wu1_ref[...])
                     + _mm(agg, wu2_ref[...]) + bu_ref[...], 0.0)
    h2_ref[...] = h2
    p_ref[...] = _mm(h2, wa_ref[...]) + bm_ref[...]
    q_ref[...] = _mm(h2, wb_ref[...])


def _t_body(e_ref, wc2_ref, t_ref):
    t_ref[...] = _mm(e_ref[...], wc2_ref[...])


def _readout_body(pp_ref, cc_ref, w1_ref, b1_ref, w2_ref, b2_ref, o_ref):
    pooled_sum = pp_ref[0] + pp_ref[1]
    counts = cc_ref[0] + cc_ref[1]
    pooled = pooled_sum / jnp.maximum(counts[:, 0:1], 1.0)
    hh = jnp.maximum(_mm(pooled, w1_ref[...]) + b1_ref[...], 0.0)
    o_ref[...] = _mm(hh, w2_ref[...]) + b2_ref[...]


def _sds(shape):
    return jax.ShapeDtypeStruct(shape, _f32)


_node0 = pl.pallas_call(
    _node0_body,
    out_shape=(_sds((N, H)), _sds((N, H)), _sds((N, H))))

_EBH = 16000

_edge0 = pl.pallas_call(
    _edge0_body,
    grid=(EH // _EBH,),
    in_specs=[
        pl.BlockSpec((D_EDGE, _EBH), lambda i: (0, i)),
        pl.BlockSpec((D_EDGE, _EBH), lambda i: (0, i + EH // _EBH)),
        pl.BlockSpec((D_EDGE, H), lambda i: (0, 0)),
        pl.BlockSpec((1, H), lambda i: (0, 0)),
        pl.BlockSpec((H, H), lambda i: (0, 0)),
    ],
    out_specs=pl.BlockSpec((_EBH, 128), lambda i: (i, 0)),
    out_shape=_sds((EH, 128)))

_upd = pl.pallas_call(
    _upd_body,
    out_shape=(_sds((N, H)), _sds((N, H)), _sds((N, H))))

_tmat = pl.pallas_call(
    _t_body,
    grid=(E // EB,),
    in_specs=[
        pl.BlockSpec((EB // 2, 128), lambda i: (i, 0)),
        pl.BlockSpec((128, 128), lambda i: (0, 0)),
    ],
    out_specs=pl.BlockSpec((EB // 2, 128), lambda i: (i, 0)),
    out_shape=_sds((EH, 128)))

_readout = pl.pallas_call(
    _readout_body,
    out_shape=_sds((G, OUT)))


_MESH = plsc.VectorSubcoreMesh(core_axis_name="c", subcore_axis_name="s",
                               num_cores=NC, num_subcores=NS)

_SC_PARAMS = pltpu.CompilerParams(use_tc_tiling_on_sc=False,
                                  needs_layout_passes=False)


def _zero_rows(ref, rows):
    def body(r, _):
        for jj in range(H // 16):
            ref[r, pl.ds(jj * 16, 16)] = jnp.zeros((16,), _f32)
        return 0
    lax.fori_loop(0, rows, body, 0)


def _sc_layer_body(t_hbm, p_hbm, q_hbm, idx_hbm,
                   e_hbm, agg_hbm,
                   t_v, gp_a, gp_b, gq_a, gq_b, idx_a, idx_b, zero_v, agg_sh,
                   semt, seme, semga, semgb, semsa, semsb):
    cid = lax.axis_index("c")
    sid = lax.axis_index("s")
    wid = sid * NC + cid

    _zero_rows(zero_v, ZR)
    nz = ROWS_PS // ZR + jnp.where(sid == NS - 1, 1, 0)

    def zcp(m, _):
        pltpu.sync_copy(zero_v,
                        agg_sh.at[pl.ds(sid * ROWS_PS + m * ZR, ZR)])
        return 0
    lax.fori_loop(0, nz, zcp, 0)
    plsc.subcore_barrier()

    nch = CH_FULL + jnp.where(wid < CH_EXTRA, 1, 0)

    def start_gathers(idxv, gpv, gqv, sem):
        pltpu.async_copy(p_hbm.at[idxv.at[0]], gpv.at[pl.ds(0, SB)], sem)
        pltpu.async_copy(p_hbm.at[idxv.at[1]], gpv.at[pl.ds(SB, SB)], sem)
        pltpu.async_copy(q_hbm.at[idxv.at[2]], gqv.at[pl.ds(0, SB)], sem)
        pltpu.async_copy(q_hbm.at[idxv.at[3]], gqv.at[pl.ds(SB, SB)], sem)

    def drain_gathers(idxv, gpv, gqv, sem):
        pltpu.make_async_copy(p_hbm.at[idxv.at[0]],
                              gpv.at[pl.ds(0, SB)], sem).wait()
        pltpu.make_async_copy(p_hbm.at[idxv.at[1]],
                              gpv.at[pl.ds(SB, SB)], sem).wait()
        pltpu.make_async_copy(q_hbm.at[idxv.at[2]],
                              gqv.at[pl.ds(0, SB)], sem).wait()
        pltpu.make_async_copy(q_hbm.at[idxv.at[3]],
                              gqv.at[pl.ds(SB, SB)], sem).wait()

    def drain_scatters(idxv, gpv, sem):
        pltpu.make_async_copy(gpv.at[pl.ds(0, SB)],
                              agg_sh.at[idxv.at[2]], sem).wait()
        pltpu.make_async_copy(gpv.at[pl.ds(SB, SB)],
                              agg_sh.at[idxv.at[3]], sem).wait()

    pltpu.sync_copy(idx_hbm.at[wid], idx_a)
    start_gathers(idx_a, gp_a, gq_a, semga)

    sets = ((gp_a, gq_a, idx_a, semga, semsa),
            (gp_b, gq_b, idx_b, semgb, semsb))

    def chunk_body(i, gpx, gqx, idxx, semgx, semsx,
                   gpy, gqy, idxy, semgy, semsy):
        baseh = (wid + i * NW) * CHH

        @pl.when(i >= 1)
        def _():
            pltpu.make_async_copy(t_v, e_hbm.at[pl.ds(0, CHH)],
                                  seme).wait()
        tcp = pltpu.async_copy(t_hbm.at[pl.ds(baseh, CHH)], t_v, semt)

        @pl.when(i >= 1)
        def _():
            drain_scatters(idxy, gpy, semsy)

        @pl.when(i + 1 < nch)
        def _():
            pltpu.sync_copy(idx_hbm.at[wid + (i + 1) * NW], idxy)
            start_gathers(idxy, gpy, gqy, semgy)

        drain_gathers(idxx, gpx, gqx, semgx)
        tcp.wait()

        @plsc.parallel_loop(0, CHH, step=1, unroll=2)
        def rows(r):
            for half in range(2):
                for jj in range(4):
                    sv = pl.ds((half * 4 + jj) * 16, 16)
                    sg = pl.ds(jj * 16, 16)
                    v = jnp.maximum(
                        t_v[r, sv] + gpx[half * SB + r, sg]
                        + gqx[half * SB + r, sg], 0.0)
                    t_v[r, sv] = v
                    gpx[half * SB + r, sg] = v

        pltpu.async_copy(t_v, e_hbm.at[pl.ds(baseh, CHH)], seme)
        pltpu.async_copy(gpx.at[pl.ds(0, SB)],
                         agg_sh.at[idxx.at[2]], semsx, add=True)
        pltpu.async_copy(gpx.at[pl.ds(SB, SB)],
                         agg_sh.at[idxx.at[3]], semsx, add=True)

    def pair(u, _):
        for x in range(2):
            i = u * 2 + x

            @pl.when(i < nch)
            def _():
                chunk_body(i, *sets[x], *sets[1 - x])
        return 0
    lax.fori_loop(0, (CH_FULL + 2) // 2, pair, 0)

    pltpu.make_async_copy(t_v, e_hbm.at[pl.ds(0, CHH)], seme).wait()

    @pl.when(nch % 2 == 1)
    def _():
        drain_scatters(idx_a, gp_a, semsa)

    @pl.when(nch % 2 == 0)
    def _():
        drain_scatters(idx_b, gp_b, semsb)

    plsc.subcore_barrier()

    @pl.when(sid < NS - 1)
    def _():
        pltpu.sync_copy(agg_sh.at[pl.ds(sid * ROWS_PS, ROWS_PS)],
                        agg_hbm.at[cid].at[pl.ds(sid * ROWS_PS, ROWS_PS)])

    @pl.when(sid == NS - 1)
    def _():
        pltpu.sync_copy(
            agg_sh.at[pl.ds((NS - 1) * ROWS_PS, N - (NS - 1) * ROWS_PS)],
            agg_hbm.at[cid].at[pl.ds((NS - 1) * ROWS_PS,
                                     N - (NS - 1) * ROWS_PS)])


_sc_layer = pl.kernel(
    _sc_layer_body,
    out_type=(jax.ShapeDtypeStruct((EH, 128), _f32),
              jax.ShapeDtypeStruct((NC, N, H), _f32)),
    mesh=_MESH,
    compiler_params=_SC_PARAMS,
    scratch_types=[
        pltpu.VMEM((CHH, 128), _f32),
        pltpu.VMEM((CH, H), _f32),
        pltpu.VMEM((CH, H), _f32),
        pltpu.VMEM((CH, H), _f32),
        pltpu.VMEM((CH, H), _f32),
        pltpu.VMEM((4, SB), jnp.int32),
        pltpu.VMEM((4, SB), jnp.int32),
        pltpu.VMEM((ZR, H), _f32),
        pltpu.VMEM_SHARED((N, H), _f32),
        pltpu.SemaphoreType.DMA,
        pltpu.SemaphoreType.DMA,
        pltpu.SemaphoreType.DMA,
        pltpu.SemaphoreType.DMA,
        pltpu.SemaphoreType.DMA,
        pltpu.SemaphoreType.DMA,
    ])


def _sc_final_body(t_hbm, p_hbm, q_hbm, idx_hbm, batch_hbm,
                   pool_hbm, cnt_hbm,
                   t_v, gp_a, gp_b, gq_a, gq_b, idx_a, idx_b, gidx_a, gidx_b,
                   batch_v, ones_v, z16_v,
                   pool_sh, cnt_sh, semt, semga, semgb, semsa, semsb):
    cid = lax.axis_index("c")
    sid = lax.axis_index("s")
    wid = sid * NC + cid

    pltpu.sync_copy(batch_hbm, batch_v)

    def ones_rows(r, _):
        ones_v[r, pl.ds(0, 16)] = jnp.ones((16,), _f32)
        return 0
    lax.fori_loop(0, SB, ones_rows, 0)
    _zero_rows(gp_a, 1)
    z16_v[0, pl.ds(0, 16)] = jnp.zeros((16,), _f32)
    pltpu.sync_copy(gp_a.at[pl.ds(0, 1)], pool_sh.at[pl.ds(sid, 1)])
    pltpu.sync_copy(z16_v, cnt_sh.at[pl.ds(sid, 1)])
    plsc.subcore_barrier()

    nch = CH_FULL + jnp.where(wid < CH_EXTRA, 1, 0)

    def start_gathers(idxv, gpv, gqv, sem):
        pltpu.async_copy(p_hbm.at[idxv.at[0]], gpv.at[pl.ds(0, SB)], sem)
        pltpu.async_copy(p_hbm.at[idxv.at[1]], gpv.at[pl.ds(SB, SB)], sem)
        pltpu.async_copy(q_hbm.at[idxv.at[2]], gqv.at[pl.ds(0, SB)], sem)
        pltpu.async_copy(q_hbm.at[idxv.at[3]], gqv.at[pl.ds(SB, SB)], sem)

    def drain_gathers(idxv, gpv, gqv, sem):
        pltpu.make_async_copy(p_hbm.at[idxv.at[0]],
                              gpv.at[pl.ds(0, SB)], sem).wait()
        pltpu.make_async_copy(p_hbm.at[idxv.at[1]],
                              gpv.at[pl.ds(SB, SB)], sem).wait()
        pltpu.make_async_copy(q_hbm.at[idxv.at[2]],
                              gqv.at[pl.ds(0, SB)], sem).wait()
        pltpu.make_async_copy(q_hbm.at[idxv.at[3]],
                              gqv.at[pl.ds(SB, SB)], sem).wait()

    def start_scatters(gidxv, gpv, sem):
        pltpu.async_copy(gpv.at[pl.ds(0, SB)],
                         pool_sh.at[gidxv.at[0]], sem, add=True)
        pltpu.async_copy(gpv.at[pl.ds(SB, SB)],
                         pool_sh.at[gidxv.at[1]], sem, add=True)
        pltpu.async_copy(ones_v, cnt_sh.at[gidxv.at[0]], sem, add=True)
        pltpu.async_copy(ones_v, cnt_sh.at[gidxv.at[1]], sem, add=True)

    def drain_scatters(gidxv, gpv, sem):
        pltpu.make_async_copy(gpv.at[pl.ds(0, SB)],
                              pool_sh.at[gidxv.at[0]], sem).wait()
        pltpu.make_async_copy(gpv.at[pl.ds(SB, SB)],
                              pool_sh.at[gidxv.at[1]], sem).wait()
        pltpu.make_async_copy(ones_v, cnt_sh.at[gidxv.at[0]], sem).wait()
        pltpu.make_async_copy(ones_v, cnt_sh.at[gidxv.at[1]], sem).wait()

    pltpu.sync_copy(idx_hbm.at[wid], idx_a)
    start_gathers(idx_a, gp_a, gq_a, semga)

    sets = ((gp_a, gq_a, idx_a, gidx_a, semga, semsa),
            (gp_b, gq_b, idx_b, gidx_b, semgb, semsb))

    def chunk_body(i, gpx, gqx, idxx, gidxx, semgx, semsx,
                   gpy, gqy, idxy, gidxy, semgy, semsy):
        baseh = (wid + i * NW) * CHH
        tcp = pltpu.async_copy(t_hbm.at[pl.ds(baseh, CHH)], t_v, semt)

        for half in range(2):
            for m in range(SB // 16):
                s = pl.ds(m * 16, 16)
                gidxx[half, s] = plsc.load_gather(batch_v, [idxx[half, s]])

        @pl.when(i >= 1)
        def _():
            drain_scatters(gidxy, gpy, semsy)

        @pl.when(i + 1 < nch)
        def _():
            pltpu.sync_copy(idx_hbm.at[wid + (i + 1) * NW], idxy)
            start_gathers(idxy, gpy, gqy, semgy)

        drain_gathers(idxx, gpx, gqx, semgx)
        tcp.wait()

        @plsc.parallel_loop(0, CHH, step=1, unroll=2)
        def rows(r):
            for half in range(2):
                for jj in range(4):
                    sv = pl.ds((half * 4 + jj) * 16, 16)
                    sg = pl.ds(jj * 16, 16)
                    gpx[half * SB + r, sg] = jnp.maximum(
                        t_v[r, sv] + gpx[half * SB + r, sg]
                        + gqx[half * SB + r, sg], 0.0)

        start_scatters(gidxx, gpx, semsx)

    def pair(u, _):
        for x in range(2):
            i = u * 2 + x

            @pl.when(i < nch)
            def _():
                chunk_body(i, *sets[x], *sets[1 - x])
        return 0
    lax.fori_loop(0, (CH_FULL + 2) // 2, pair, 0)

    @pl.when(nch % 2 == 1)
    def _():
        drain_scatters(gidx_a, gp_a, semsa)

    @pl.when(nch % 2 == 0)
    def _():
        drain_scatters(gidx_b, gp_b, semsb)

    plsc.subcore_barrier()

    @pl.when(sid == 0)
    def _():
        pltpu.sync_copy(pool_sh, pool_hbm.at[cid])
        pltpu.sync_copy(cnt_sh, cnt_hbm.at[cid])


_sc_final = pl.kernel(
    _sc_final_body,
    out_type=(jax.ShapeDtypeStruct((NC, G, H), _f32),
              jax.ShapeDtypeStruct((NC, G, 16), _f32)),
    mesh=_MESH,
    compiler_params=_SC_PARAMS,
    scratch_types=[
        pltpu.VMEM((CHH, 128), _f32),
        pltpu.VMEM((CH, H), _f32),
        pltpu.VMEM((CH, H), _f32),
        pltpu.VMEM((CH, H), _f32),
        pltpu.VMEM((CH, H), _f32),
        pltpu.VMEM((4, SB), jnp.int32),
        pltpu.VMEM((4, SB), jnp.int32),
        pltpu.VMEM((2, SB), jnp.int32),
        pltpu.VMEM((2, SB), jnp.int32),
        pltpu.VMEM((N,), jnp.int32),
        pltpu.VMEM((SB, 16), _f32),
        pltpu.VMEM((1, 16), _f32),
        pltpu.VMEM_SHARED((G, H), _f32),
        pltpu.VMEM_SHARED((G, 16), _f32),
        pltpu.SemaphoreType.DMA,
        pltpu.SemaphoreType.DMA,
        pltpu.SemaphoreType.DMA,
        pltpu.SemaphoreType.DMA,
        pltpu.SemaphoreType.DMA,
    ])


def kernel(x, edge_attr, params, edge_index, batch):
    src, dst = edge_index[0], edge_index[1]
    idx = jnp.stack([src[:EH].reshape(NCHUNKS, SB),
                     src[EH:].reshape(NCHUNKS, SB),
                     dst[:EH].reshape(NCHUNKS, SB),
                     dst[EH:].reshape(NCHUNKS, SB)], axis=1)

    def msplit(l):
        w = params[f'W_msg_{l}']
        return w[:H], w[H:2 * H], w[2 * H:]

    def usplit(l):
        w = params[f'W_upd_{l}']
        return w[:H], w[H:]

    def b2d(b):
        return b.reshape(1, H)

    def blockdiag2(w):
        z = jnp.zeros((H, H), _f32)
        return jnp.concatenate([jnp.concatenate([w, z], axis=1),
                                jnp.concatenate([z, w], axis=1)], axis=0)

    wa0, wb0, wc0 = msplit(0)
    h, p, q = _node0(x, params['Wn_enc'], b2d(params['bn_enc']),
                     wa0, wb0, b2d(params['b_msg_0']))
    eat = edge_attr.T
    t = _edge0(eat, eat, params['We_enc'], b2d(params['be_enc']), wc0)

    for l in range(2):
        e, aggp = _sc_layer(t, p, q, idx)
        wu1, wu2 = usplit(l)
        wa, wb, wc = msplit(l + 1)
        h, p, q = _upd(h, aggp, wu1, wu2, b2d(params[f'b_upd_{l}']),
                       wa, wb, b2d(params[f'b_msg_{l + 1}']))
        t = _tmat(e, blockdiag2(wc))

    poolp, cntp = _sc_final(t, p, q, idx, batch)
    out = _readout(poolp, cntp, params['W_r1'], b2d(params['b_r1']),
                   params['W_r2'], b2d(params['b_r2']))
    return out


# speedup vs baseline: 11.6217x; 1.0009x over previous
import jax
import jax.numpy as jnp
from jax import lax
from jax.experimental import pallas as pl
from jax.experimental.pallas import tpu as pltpu
from jax.experimental.pallas import tpu_sc as plsc

N = 10000
E = 320000
D_IN = 128
D_EDGE = 16
H = 64
OUT = 64
G = 16

NC, NS = 2, 16
NW = NC * NS
CH = 256
CHH = CH // 2
SB = 128
NCHUNKS = E // CH
CH_FULL = NCHUNKS // NW
CH_EXTRA = NCHUNKS - CH_FULL * NW
ROWS_PS = 624
ZR = 16
EB = 8000
EH = E // 2

_f32 = jnp.float32


def _mm(a, b):
    return lax.dot_general(a, b, (((1,), (0,)), ((), ())),
                           preferred_element_type=jnp.float32)


def _node0_body(x_ref, wn_ref, bn_ref, wa_ref, wb_ref, bm_ref,
                h_ref, p_ref, q_ref):
    h = jnp.maximum(_mm(x_ref[...], wn_ref[...]) + bn_ref[...], 0.0)
    h_ref[...] = h
    p_ref[...] = _mm(h, wa_ref[...]) + bm_ref[...]
    q_ref[...] = _mm(h, wb_ref[...])


def _dotT(a, b):
    return lax.dot_general(a, b, (((0,), (0,)), ((), ())),
                           preferred_element_type=jnp.float32)


def _edge0_body(lo_ref, hi_ref, we_ref, be_ref, wc_ref, t_ref):
    lo = jnp.maximum(_dotT(lo_ref[...], we_ref[...]) + be_ref[...], 0.0)
    hi = jnp.maximum(_dotT(hi_ref[...], we_ref[...]) + be_ref[...], 0.0)
    t_ref[...] = jnp.concatenate(
        [_mm(lo, wc_ref[...]), _mm(hi, wc_ref[...])], axis=1)


def _upd_body(h_ref, agg_ref, wu1_ref, wu2_ref, bu_ref,
              wa_ref, wb_ref, bm_ref, h2_ref, p_ref, q_ref):
    agg = agg_ref[0] + agg_ref[1]
    h2 = jnp.maximum(_mm(h_ref[...], wu1_ref[...])
                     + _mm(agg, wu2_ref[...]) + bu_ref[...], 0.0)
    h2_ref[...] = h2
    p_ref[...] = _mm(h2, wa_ref[...]) + bm_ref[...]
    q_ref[...] = _mm(h2, wb_ref[...])


def _t_body(e_ref, wc2_ref, t_ref):
    t_ref[...] = _mm(e_ref[...], wc2_ref[...])


def _readout_body(pp_ref, cc_ref, w1_ref, b1_ref, w2_ref, b2_ref, o_ref):
    pooled_sum = pp_ref[0] + pp_ref[1]
    counts = cc_ref[0] + cc_ref[1]
    pooled = pooled_sum / jnp.maximum(counts[:, 0:1], 1.0)
    hh = jnp.maximum(_mm(pooled, w1_ref[...]) + b1_ref[...], 0.0)
    o_ref[...] = _mm(hh, w2_ref[...]) + b2_ref[...]


def _sds(shape):
    return jax.ShapeDtypeStruct(shape, _f32)


_node0 = pl.pallas_call(
    _node0_body,
    out_shape=(_sds((N, H)), _sds((N, H)), _sds((N, H))))

_EBH = 16000

_edge0 = pl.pallas_call(
    _edge0_body,
    grid=(EH // _EBH,),
    in_specs=[
        pl.BlockSpec((D_EDGE, _EBH), lambda i: (0, i)),
        pl.BlockSpec((D_EDGE, _EBH), lambda i: (0, i + EH // _EBH)),
        pl.BlockSpec((D_EDGE, H), lambda i: (0, 0)),
        pl.BlockSpec((1, H), lambda i: (0, 0)),
        pl.BlockSpec((H, H), lambda i: (0, 0)),
    ],
    out_specs=pl.BlockSpec((_EBH, 128), lambda i: (i, 0)),
    out_shape=_sds((EH, 128)))

_upd = pl.pallas_call(
    _upd_body,
    out_shape=(_sds((N, H)), _sds((N, H)), _sds((N, H))))

_tmat = pl.pallas_call(
    _t_body,
    grid=(E // EB,),
    in_specs=[
        pl.BlockSpec((EB // 2, 128), lambda i: (i, 0)),
        pl.BlockSpec((128, 128), lambda i: (0, 0)),
    ],
    out_specs=pl.BlockSpec((EB // 2, 128), lambda i: (i, 0)),
    out_shape=_sds((EH, 128)))

_readout = pl.pallas_call(
    _readout_body,
    out_shape=_sds((G, OUT)))


_MESH = plsc.VectorSubcoreMesh(core_axis_name="c", subcore_axis_name="s",
                               num_cores=NC, num_subcores=NS)

_SC_PARAMS = pltpu.CompilerParams(use_tc_tiling_on_sc=False,
                                  needs_layout_passes=False)


def _zero_rows(ref, rows):
    def body(r, _):
        for jj in range(H // 16):
            ref[r, pl.ds(jj * 16, 16)] = jnp.zeros((16,), _f32)
        return 0
    lax.fori_loop(0, rows, body, 0)


def _sc_layer_body(t_hbm, p_hbm, q_hbm, idx_hbm,
                   e_hbm, agg_hbm,
                   t_v, gp_a, gp_b, gq_a, gq_b, idx_a, idx_b, zero_v, agg_sh,
                   semt, seme, semga, semgb, semsa, semsb):
    cid = lax.axis_index("c")
    sid = lax.axis_index("s")
    wid = sid * NC + cid

    _zero_rows(zero_v, ZR)
    nz = ROWS_PS // ZR + jnp.where(sid == NS - 1, 1, 0)

    def zcp(m, _):
        pltpu.sync_copy(zero_v,
                        agg_sh.at[pl.ds(sid * ROWS_PS + m * ZR, ZR)])
        return 0
    lax.fori_loop(0, nz, zcp, 0)
    plsc.subcore_barrier()

    nch = CH_FULL + jnp.where(wid < CH_EXTRA, 1, 0)

    def start_gathers(idxv, gpv, gqv, sem):
        pltpu.async_copy(p_hbm.at[idxv.at[0]], gpv.at[pl.ds(0, SB)], sem)
        pltpu.async_copy(p_hbm.at[idxv.at[1]], gpv.at[pl.ds(SB, SB)], sem)
        pltpu.async_copy(q_hbm.at[idxv.at[2]], gqv.at[pl.ds(0, SB)], sem)
        pltpu.async_copy(q_hbm.at[idxv.at[3]], gqv.at[pl.ds(SB, SB)], sem)

    def drain_gathers(idxv, gpv, gqv, sem):
        pltpu.make_async_copy(p_hbm.at[idxv.at[0]],
                              gpv.at[pl.ds(0, SB)], sem).wait()
        pltpu.make_async_copy(p_hbm.at[idxv.at[1]],
                              gpv.at[pl.ds(SB, SB)], sem).wait()
        pltpu.make_async_copy(q_hbm.at[idxv.at[2]],
                              gqv.at[pl.ds(0, SB)], sem).wait()
        pltpu.make_async_copy(q_hbm.at[idxv.at[3]],
                              gqv.at[pl.ds(SB, SB)], sem).wait()

    def drain_scatters(idxv, gpv, sem):
        pltpu.make_async_copy(gpv.at[pl.ds(0, SB)],
                              agg_sh.at[idxv.at[2]], sem).wait()
        pltpu.make_async_copy(gpv.at[pl.ds(SB, SB)],
                              agg_sh.at[idxv.at[3]], sem).wait()

    pltpu.sync_copy(idx_hbm.at[wid], idx_a)
    start_gathers(idx_a, gp_a, gq_a, semga)

    sets = ((gp_a, gq_a, idx_a, semga, semsa),
            (gp_b, gq_b, idx_b, semgb, semsb))

    def chunk_body(i, gpx, gqx, idxx, semgx, semsx,
                   gpy, gqy, idxy, semgy, semsy):
        baseh = (wid + i * NW) * CHH

        @pl.when(i >= 1)
        def _():
            pltpu.make_async_copy(t_v, e_hbm.at[pl.ds(0, CHH)],
                                  seme).wait()
        tcp = pltpu.async_copy(t_hbm.at[pl.ds(baseh, CHH)], t_v, semt)

        @pl.when(i >= 1)
        def _():
            drain_scatters(idxy, gpy, semsy)

        @pl.when(i + 1 < nch)
        def _():
            pltpu.sync_copy(idx_hbm.at[wid + (i + 1) * NW], idxy)
            start_gathers(idxy, gpy, gqy, semgy)

        drain_gathers(idxx, gpx, gqx, semgx)
        tcp.wait()

        @plsc.parallel_loop(0, CHH, step=1, unroll=4)
        def rows(r):
            for half in range(2):
                for jj in range(4):
                    sv = pl.ds((half * 4 + jj) * 16, 16)
                    sg = pl.ds(jj * 16, 16)
                    v = jnp.maximum(
                        t_v[r, sv] + gpx[half * SB + r, sg]
                        + gqx[half * SB + r, sg], 0.0)
                    t_v[r, sv] = v
                    gpx[half * SB + r, sg] = v

        pltpu.async_copy(t_v, e_hbm.at[pl.ds(baseh, CHH)], seme)
        pltpu.async_copy(gpx.at[pl.ds(0, SB)],
                         agg_sh.at[idxx.at[2]], semsx, add=True)
        pltpu.async_copy(gpx.at[pl.ds(SB, SB)],
                         agg_sh.at[idxx.at[3]], semsx, add=True)

    def pair(u, _):
        for x in range(2):
            i = u * 2 + x

            @pl.when(i < nch)
            def _():
                chunk_body(i, *sets[x], *sets[1 - x])
        return 0
    lax.fori_loop(0, (CH_FULL + 2) // 2, pair, 0)

    pltpu.make_async_copy(t_v, e_hbm.at[pl.ds(0, CHH)], seme).wait()

    @pl.when(nch % 2 == 1)
    def _():
        drain_scatters(idx_a, gp_a, semsa)

    @pl.when(nch % 2 == 0)
    def _():
        drain_scatters(idx_b, gp_b, semsb)

    plsc.subcore_barrier()

    @pl.when(sid < NS - 1)
    def _():
        pltpu.sync_copy(agg_sh.at[pl.ds(sid * ROWS_PS, ROWS_PS)],
                        agg_hbm.at[cid].at[pl.ds(sid * ROWS_PS, ROWS_PS)])

    @pl.when(sid == NS - 1)
    def _():
        pltpu.sync_copy(
            agg_sh.at[pl.ds((NS - 1) * ROWS_PS, N - (NS - 1) * ROWS_PS)],
            agg_hbm.at[cid].at[pl.ds((NS - 1) * ROWS_PS,
                                     N - (NS - 1) * ROWS_PS)])


_sc_layer = pl.kernel(
    _sc_layer_body,
    out_type=(jax.ShapeDtypeStruct((EH, 128), _f32),
              jax.ShapeDtypeStruct((NC, N, H), _f32)),
    mesh=_MESH,
    compiler_params=_SC_PARAMS,
    scratch_types=[
        pltpu.VMEM((CHH, 128), _f32),
        pltpu.VMEM((CH, H), _f32),
        pltpu.VMEM((CH, H), _f32),
        pltpu.VMEM((CH, H), _f32),
        pltpu.VMEM((CH, H), _f32),
        pltpu.VMEM((4, SB), jnp.int32),
        pltpu.VMEM((4, SB), jnp.int32),
        pltpu.VMEM((ZR, H), _f32),
        pltpu.VMEM_SHARED((N, H), _f32),
        pltpu.SemaphoreType.DMA,
        pltpu.SemaphoreType.DMA,
        pltpu.SemaphoreType.DMA,
        pltpu.SemaphoreType.DMA,
        pltpu.SemaphoreType.DMA,
        pltpu.SemaphoreType.DMA,
    ])


def _sc_final_body(t_hbm, p_hbm, q_hbm, idx_hbm, batch_hbm,
                   pool_hbm, cnt_hbm,
                   t_v, gp_a, gp_b, gq_a, gq_b, idx_a, idx_b, gidx_a, gidx_b,
                   batch_v, ones_v, z16_v,
                   pool_sh, cnt_sh, semt, semga, semgb, semsa, semsb):
    cid = lax.axis_index("c")
    sid = lax.axis_index("s")
    wid = sid * NC + cid

    pltpu.sync_copy(batch_hbm, batch_v)

    def ones_rows(r, _):
        ones_v[r, pl.ds(0, 16)] = jnp.ones((16,), _f32)
        return 0
    lax.fori_loop(0, SB, ones_rows, 0)
    _zero_rows(gp_a, 1)
    z16_v[0, pl.ds(0, 16)] = jnp.zeros((16,), _f32)
    pltpu.sync_copy(gp_a.at[pl.ds(0, 1)], pool_sh.at[pl.ds(sid, 1)])
    pltpu.sync_copy(z16_v, cnt_sh.at[pl.ds(sid, 1)])
    plsc.subcore_barrier()

    nch = CH_FULL + jnp.where(wid < CH_EXTRA, 1, 0)

    def start_gathers(idxv, gpv, gqv, sem):
        pltpu.async_copy(p_hbm.at[idxv.at[0]], gpv.at[pl.ds(0, SB)], sem)
        pltpu.async_copy(p_hbm.at[idxv.at[1]], gpv.at[pl.ds(SB, SB)], sem)
        pltpu.async_copy(q_hbm.at[idxv.at[2]], gqv.at[pl.ds(0, SB)], sem)
        pltpu.async_copy(q_hbm.at[idxv.at[3]], gqv.at[pl.ds(SB, SB)], sem)

    def drain_gathers(idxv, gpv, gqv, sem):
        pltpu.make_async_copy(p_hbm.at[idxv.at[0]],
                              gpv.at[pl.ds(0, SB)], sem).wait()
        pltpu.make_async_copy(p_hbm.at[idxv.at[1]],
                              gpv.at[pl.ds(SB, SB)], sem).wait()
        pltpu.make_async_copy(q_hbm.at[idxv.at[2]],
                              gqv.at[pl.ds(0, SB)], sem).wait()
        pltpu.make_async_copy(q_hbm.at[idxv.at[3]],
                              gqv.at[pl.ds(SB, SB)], sem).wait()

    def start_scatters(gidxv, gpv, sem):
        pltpu.async_copy(gpv.at[pl.ds(0, SB)],
                         pool_sh.at[gidxv.at[0]], sem, add=True)
        pltpu.async_copy(gpv.at[pl.ds(SB, SB)],
                         pool_sh.at[gidxv.at[1]], sem, add=True)
        pltpu.async_copy(ones_v, cnt_sh.at[gidxv.at[0]], sem, add=True)
        pltpu.async_copy(ones_v, cnt_sh.at[gidxv.at[1]], sem, add=True)

    def drain_scatters(gidxv, gpv, sem):
        pltpu.make_async_copy(gpv.at[pl.ds(0, SB)],
                              pool_sh.at[gidxv.at[0]], sem).wait()
        pltpu.make_async_copy(gpv.at[pl.ds(SB, SB)],
                              pool_sh.at[gidxv.at[1]], sem).wait()
        pltpu.make_async_copy(ones_v, cnt_sh.at[gidxv.at[0]], sem).wait()
        pltpu.make_async_copy(ones_v, cnt_sh.at[gidxv.at[1]], sem).wait()

    pltpu.sync_copy(idx_hbm.at[wid], idx_a)
    start_gathers(idx_a, gp_a, gq_a, semga)

    sets = ((gp_a, gq_a, idx_a, gidx_a, semga, semsa),
            (gp_b, gq_b, idx_b, gidx_b, semgb, semsb))

    def chunk_body(i, gpx, gqx, idxx, gidxx, semgx, semsx,
                   gpy, gqy, idxy, gidxy, semgy, semsy):
        baseh = (wid + i * NW) * CHH
        tcp = pltpu.async_copy(t_hbm.at[pl.ds(baseh, CHH)], t_v, semt)

        for half in range(2):
            for m in range(SB // 16):
                s = pl.ds(m * 16, 16)
                gidxx[half, s] = plsc.load_gather(batch_v, [idxx[half, s]])

        @pl.when(i >= 1)
        def _():
            drain_scatters(gidxy, gpy, semsy)

        @pl.when(i + 1 < nch)
        def _():
            pltpu.sync_copy(idx_hbm.at[wid + (i + 1) * NW], idxy)
            start_gathers(idxy, gpy, gqy, semgy)

        drain_gathers(idxx, gpx, gqx, semgx)
        tcp.wait()

        @plsc.parallel_loop(0, CHH, step=1, unroll=4)
        def rows(r):
            for half in range(2):
                for jj in range(4):
                    sv = pl.ds((half * 4 + jj) * 16, 16)
                    sg = pl.ds(jj * 16, 16)
                    gpx[half * SB + r, sg] = jnp.maximum(
                        t_v[r, sv] + gpx[half * SB + r, sg]
                        + gqx[half * SB + r, sg], 0.0)

        start_scatters(gidxx, gpx, semsx)

    def pair(u, _):
        for x in range(2):
            i = u * 2 + x

            @pl.when(i < nch)
            def _():
                chunk_body(i, *sets[x], *sets[1 - x])
        return 0
    lax.fori_loop(0, (CH_FULL + 2) // 2, pair, 0)

    @pl.when(nch % 2 == 1)
    def _():
        drain_scatters(gidx_a, gp_a, semsa)

    @pl.when(nch % 2 == 0)
    def _():
        drain_scatters(gidx_b, gp_b, semsb)

    plsc.subcore_barrier()

    @pl.when(sid == 0)
    def _():
        pltpu.sync_copy(pool_sh, pool_hbm.at[cid])
        pltpu.sync_copy(cnt_sh, cnt_hbm.at[cid])


_sc_final = pl.kernel(
    _sc_final_body,
    out_type=(jax.ShapeDtypeStruct((NC, G, H), _f32),
              jax.ShapeDtypeStruct((NC, G, 16), _f32)),
    mesh=_MESH,
    compiler_params=_SC_PARAMS,
    scratch_types=[
        pltpu.VMEM((CHH, 128), _f32),
        pltpu.VMEM((CH, H), _f32),
        pltpu.VMEM((CH, H), _f32),
        pltpu.VMEM((CH, H), _f32),
        pltpu.VMEM((CH, H), _f32),
        pltpu.VMEM((4, SB), jnp.int32),
        pltpu.VMEM((4, SB), jnp.int32),
        pltpu.VMEM((2, SB), jnp.int32),
        pltpu.VMEM((2, SB), jnp.int32),
        pltpu.VMEM((N,), jnp.int32),
        pltpu.VMEM((SB, 16), _f32),
        pltpu.VMEM((1, 16), _f32),
        pltpu.VMEM_SHARED((G, H), _f32),
        pltpu.VMEM_SHARED((G, 16), _f32),
        pltpu.SemaphoreType.DMA,
        pltpu.SemaphoreType.DMA,
        pltpu.SemaphoreType.DMA,
        pltpu.SemaphoreType.DMA,
        pltpu.SemaphoreType.DMA,
    ])


def kernel(x, edge_attr, params, edge_index, batch):
    src, dst = edge_index[0], edge_index[1]
    idx = jnp.stack([src[:EH].reshape(NCHUNKS, SB),
                     src[EH:].reshape(NCHUNKS, SB),
                     dst[:EH].reshape(NCHUNKS, SB),
                     dst[EH:].reshape(NCHUNKS, SB)], axis=1)

    def msplit(l):
        w = params[f'W_msg_{l}']
        return w[:H], w[H:2 * H], w[2 * H:]

    def usplit(l):
        w = params[f'W_upd_{l}']
        return w[:H], w[H:]

    def b2d(b):
        return b.reshape(1, H)

    def blockdiag2(w):
        z = jnp.zeros((H, H), _f32)
        return jnp.concatenate([jnp.concatenate([w, z], axis=1),
                                jnp.concatenate([z, w], axis=1)], axis=0)

    wa0, wb0, wc0 = msplit(0)
    h, p, q = _node0(x, params['Wn_enc'], b2d(params['bn_enc']),
                     wa0, wb0, b2d(params['b_msg_0']))
    eat = edge_attr.T
    t = _edge0(eat, eat, params['We_enc'], b2d(params['be_enc']), wc0)

    for l in range(2):
        e, aggp = _sc_layer(t, p, q, idx)
        wu1, wu2 = usplit(l)
        wa, wb, wc = msplit(l + 1)
        h, p, q = _upd(h, aggp, wu1, wu2, b2d(params[f'b_upd_{l}']),
                       wa, wb, b2d(params[f'b_msg_{l + 1}']))
        t = _tmat(e, blockdiag2(wc))

    poolp, cntp = _sc_final(t, p, q, idx, batch)
    out = _readout(poolp, cntp, params['W_r1'], b2d(params['b_r1']),
                   params['W_r2'], b2d(params['b_r2']))
    return out


# speedup vs baseline: 11.8450x; 1.0192x over previous
import jax
import jax.numpy as jnp
from jax import lax
from jax.experimental import pallas as pl
from jax.experimental.pallas import tpu as pltpu
from jax.experimental.pallas import tpu_sc as plsc

N = 10000
E = 320000
D_IN = 128
D_EDGE = 16
H = 64
OUT = 64
G = 16

NC, NS = 2, 16
NW = NC * NS
CH = 256
CHH = CH // 2
SB = 128
NCHUNKS = E // CH
CH_FULL = NCHUNKS // NW
CH_EXTRA = NCHUNKS - CH_FULL * NW
ROWS_PS = 624
ZR = 16
EB = 8000
EH = E // 2

_f32 = jnp.float32


def _mm(a, b):
    return lax.dot_general(a, b, (((1,), (0,)), ((), ())),
                           preferred_element_type=jnp.float32)


def _node0_body(x_ref, wn_ref, bn_ref, wa_ref, wb_ref, bm_ref,
                h_ref, p_ref, q_ref):
    h = jnp.maximum(_mm(x_ref[...], wn_ref[...]) + bn_ref[...], 0.0)
    h_ref[...] = h
    p_ref[...] = _mm(h, wa_ref[...]) + bm_ref[...]
    q_ref[...] = _mm(h, wb_ref[...])


def _dotT(a, b):
    return lax.dot_general(a, b, (((0,), (0,)), ((), ())),
                           preferred_element_type=jnp.float32)


def _edge0_body(lo_ref, hi_ref, we_ref, be_ref, wc_ref, t_ref):
    lo = jnp.maximum(_dotT(lo_ref[...], we_ref[...]) + be_ref[...], 0.0)
    hi = jnp.maximum(_dotT(hi_ref[...], we_ref[...]) + be_ref[...], 0.0)
    t_ref[...] = jnp.concatenate(
        [_mm(lo, wc_ref[...]), _mm(hi, wc_ref[...])], axis=1)


def _upd_body(h_ref, agg_ref, wu1_ref, wu2_ref, bu_ref,
              wa_ref, wb_ref, bm_ref, h2_ref, p_ref, q_ref):
    agg = agg_ref[0] + agg_ref[1]
    h2 = jnp.maximum(_mm(h_ref[...], wu1_ref[...])
                     + _mm(agg, wu2_ref[...]) + bu_ref[...], 0.0)
    h2_ref[...] = h2
    p_ref[...] = _mm(h2, wa_ref[...]) + bm_ref[...]
    q_ref[...] = _mm(h2, wb_ref[...])


def _t_body(e_ref, wc2_ref, t_ref):
    t_ref[...] = _mm(e_ref[...], wc2_ref[...])


def _readout_body(pp_ref, cc_ref, w1_ref, b1_ref, w2_ref, b2_ref, o_ref):
    pooled_sum = pp_ref[0] + pp_ref[1]
    counts = cc_ref[0] + cc_ref[1]
    pooled = pooled_sum / jnp.maximum(counts[:, 0:1], 1.0)
    hh = jnp.maximum(_mm(pooled, w1_ref[...]) + b1_ref[...], 0.0)
    o_ref[...] = _mm(hh, w2_ref[...]) + b2_ref[...]


def _sds(shape):
    return jax.ShapeDtypeStruct(shape, _f32)


_node0 = pl.pallas_call(
    _node0_body,
    out_shape=(_sds((N, H)), _sds((N, H)), _sds((N, H))))

_EBH = 16000

_edge0 = pl.pallas_call(
    _edge0_body,
    grid=(EH // _EBH,),
    compiler_params=pltpu.CompilerParams(fuse_transposed_lhs_in_matmul=True),
    in_specs=[
        pl.BlockSpec((D_EDGE, _EBH), lambda i: (0, i)),
        pl.BlockSpec((D_EDGE, _EBH), lambda i: (0, i + EH // _EBH)),
        pl.BlockSpec((D_EDGE, H), lambda i: (0, 0)),
        pl.BlockSpec((1, H), lambda i: (0, 0)),
        pl.BlockSpec((H, H), lambda i: (0, 0)),
    ],
    out_specs=pl.BlockSpec((_EBH, 128), lambda i: (i, 0)),
    out_shape=_sds((EH, 128)))

_upd = pl.pallas_call(
    _upd_body,
    out_shape=(_sds((N, H)), _sds((N, H)), _sds((N, H))))

_TB = 8000

_tmat = pl.pallas_call(
    _t_body,
    grid=(EH // _TB,),
    in_specs=[
        pl.BlockSpec((_TB, 128), lambda i: (i, 0)),
        pl.BlockSpec((128, 128), lambda i: (0, 0)),
    ],
    out_specs=pl.BlockSpec((_TB, 128), lambda i: (i, 0)),
    out_shape=_sds((EH, 128)))

_readout = pl.pallas_call(
    _readout_body,
    out_shape=_sds((G, OUT)))


_MESH = plsc.VectorSubcoreMesh(core_axis_name="c", subcore_axis_name="s",
                               num_cores=NC, num_subcores=NS)

_SC_PARAMS = pltpu.CompilerParams(use_tc_tiling_on_sc=False,
                                  needs_layout_passes=False)


def _zero_rows(ref, rows):
    def body(r, _):
        for jj in range(H // 16):
            ref[r, pl.ds(jj * 16, 16)] = jnp.zeros((16,), _f32)
        return 0
    lax.fori_loop(0, rows, body, 0)


def _sc_layer_body(t_hbm, p_hbm, q_hbm, idx_hbm,
                   e_hbm, agg_hbm,
                   t_v, gp_a, gp_b, gq_a, gq_b, idx_a, idx_b, zero_v, agg_sh,
                   semt, seme, semga, semgb, semsa, semsb):
    cid = lax.axis_index("c")
    sid = lax.axis_index("s")
    wid = sid * NC + cid

    _zero_rows(zero_v, ZR)
    nz = ROWS_PS // ZR + jnp.where(sid == NS - 1, 1, 0)

    def zcp(m, _):
        pltpu.sync_copy(zero_v,
                        agg_sh.at[pl.ds(sid * ROWS_PS + m * ZR, ZR)])
        return 0
    lax.fori_loop(0, nz, zcp, 0)
    plsc.subcore_barrier()

    nch = CH_FULL + jnp.where(wid < CH_EXTRA, 1, 0)

    def start_gathers(idxv, gpv, gqv, sem):
        pltpu.async_copy(p_hbm.at[idxv.at[0]], gpv.at[pl.ds(0, SB)], sem)
        pltpu.async_copy(p_hbm.at[idxv.at[1]], gpv.at[pl.ds(SB, SB)], sem)
        pltpu.async_copy(q_hbm.at[idxv.at[2]], gqv.at[pl.ds(0, SB)], sem)
        pltpu.async_copy(q_hbm.at[idxv.at[3]], gqv.at[pl.ds(SB, SB)], sem)

    def drain_gathers(idxv, gpv, gqv, sem):
        pltpu.make_async_copy(p_hbm.at[idxv.at[0]],
                              gpv.at[pl.ds(0, SB)], sem).wait()
        pltpu.make_async_copy(p_hbm.at[idxv.at[1]],
                              gpv.at[pl.ds(SB, SB)], sem).wait()
        pltpu.make_async_copy(q_hbm.at[idxv.at[2]],
                              gqv.at[pl.ds(0, SB)], sem).wait()
        pltpu.make_async_copy(q_hbm.at[idxv.at[3]],
                              gqv.at[pl.ds(SB, SB)], sem).wait()

    def drain_scatters(idxv, gpv, sem):
        pltpu.make_async_copy(gpv.at[pl.ds(0, SB)],
                              agg_sh.at[idxv.at[2]], sem).wait()
        pltpu.make_async_copy(gpv.at[pl.ds(SB, SB)],
                              agg_sh.at[idxv.at[3]], sem).wait()

    pltpu.sync_copy(idx_hbm.at[wid], idx_a)
    start_gathers(idx_a, gp_a, gq_a, semga)

    sets = ((gp_a, gq_a, idx_a, semga, semsa),
            (gp_b, gq_b, idx_b, semgb, semsb))

    def chunk_body(i, gpx, gqx, idxx, semgx, semsx,
                   gpy, gqy, idxy, semgy, semsy):
        baseh = (wid + i * NW) * CHH

        @pl.when(i >= 1)
        def _():
            pltpu.make_async_copy(t_v, e_hbm.at[pl.ds(0, CHH)],
                                  seme).wait()
        tcp = pltpu.async_copy(t_hbm.at[pl.ds(baseh, CHH)], t_v, semt)

        @pl.when(i >= 1)
        def _():
            drain_scatters(idxy, gpy, semsy)

        @pl.when(i + 1 < nch)
        def _():
            pltpu.sync_copy(idx_hbm.at[wid + (i + 1) * NW], idxy)
            start_gathers(idxy, gpy, gqy, semgy)

        drain_gathers(idxx, gpx, gqx, semgx)
        tcp.wait()

        @plsc.parallel_loop(0, CHH, step=1, unroll=4)
        def rows(r):
            for half in range(2):
                for jj in range(4):
                    sv = pl.ds((half * 4 + jj) * 16, 16)
                    sg = pl.ds(jj * 16, 16)
                    v = jnp.maximum(
                        t_v[r, sv] + gpx[half * SB + r, sg]
                        + gqx[half * SB + r, sg], 0.0)
                    t_v[r, sv] = v
                    gpx[half * SB + r, sg] = v

        pltpu.async_copy(t_v, e_hbm.at[pl.ds(baseh, CHH)], seme)
        pltpu.async_copy(gpx.at[pl.ds(0, SB)],
                         agg_sh.at[idxx.at[2]], semsx, add=True)
        pltpu.async_copy(gpx.at[pl.ds(SB, SB)],
                         agg_sh.at[idxx.at[3]], semsx, add=True)

    def pair(u, _):
        for x in range(2):
            i = u * 2 + x

            @pl.when(i < nch)
            def _():
                chunk_body(i, *sets[x], *sets[1 - x])
        return 0
    lax.fori_loop(0, (CH_FULL + 2) // 2, pair, 0)

    pltpu.make_async_copy(t_v, e_hbm.at[pl.ds(0, CHH)], seme).wait()

    @pl.when(nch % 2 == 1)
    def _():
        drain_scatters(idx_a, gp_a, semsa)

    @pl.when(nch % 2 == 0)
    def _():
        drain_scatters(idx_b, gp_b, semsb)

    plsc.subcore_barrier()

    @pl.when(sid < NS - 1)
    def _():
        pltpu.sync_copy(agg_sh.at[pl.ds(sid * ROWS_PS, ROWS_PS)],
                        agg_hbm.at[cid].at[pl.ds(sid * ROWS_PS, ROWS_PS)])

    @pl.when(sid == NS - 1)
    def _():
        pltpu.sync_copy(
            agg_sh.at[pl.ds((NS - 1) * ROWS_PS, N - (NS - 1) * ROWS_PS)],
            agg_hbm.at[cid].at[pl.ds((NS - 1) * ROWS_PS,
                                     N - (NS - 1) * ROWS_PS)])


_sc_layer = pl.kernel(
    _sc_layer_body,
    out_type=(jax.ShapeDtypeStruct((EH, 128), _f32),
              jax.ShapeDtypeStruct((NC, N, H), _f32)),
    mesh=_MESH,
    compiler_params=_SC_PARAMS,
    scratch_types=[
        pltpu.VMEM((CHH, 128), _f32),
        pltpu.VMEM((CH, H), _f32),
        pltpu.VMEM((CH, H), _f32),
        pltpu.VMEM((CH, H), _f32),
        pltpu.VMEM((CH, H), _f32),
        pltpu.VMEM((4, SB), jnp.int32),
        pltpu.VMEM((4, SB), jnp.int32),
        pltpu.VMEM((ZR, H), _f32),
        pltpu.VMEM_SHARED((N, H), _f32),
        pltpu.SemaphoreType.DMA,
        pltpu.SemaphoreType.DMA,
        pltpu.SemaphoreType.DMA,
        pltpu.SemaphoreType.DMA,
        pltpu.SemaphoreType.DMA,
        pltpu.SemaphoreType.DMA,
    ])


def _sc_final_body(t_hbm, p_hbm, q_hbm, idx_hbm, batch_hbm,
                   pool_hbm, cnt_hbm,
                   t_v, gp_a, gp_b, gq_a, gq_b, idx_a, idx_b, gidx_a, gidx_b,
                   batch_v, ones_v, z16_v,
                   pool_sh, cnt_sh, semt, semga, semgb, semsa, semsb):
    cid = lax.axis_index("c")
    sid = lax.axis_index("s")
    wid = sid * NC + cid

    pltpu.sync_copy(batch_hbm, batch_v)

    def ones_rows(r, _):
        ones_v[r, pl.ds(0, 16)] = jnp.ones((16,), _f32)
        return 0
    lax.fori_loop(0, SB, ones_rows, 0)
    _zero_rows(gp_a, 1)
    z16_v[0, pl.ds(0, 16)] = jnp.zeros((16,), _f32)
    pltpu.sync_copy(gp_a.at[pl.ds(0, 1)], pool_sh.at[pl.ds(sid, 1)])
    pltpu.sync_copy(z16_v, cnt_sh.at[pl.ds(sid, 1)])
    plsc.subcore_barrier()

    nch = CH_FULL + jnp.where(wid < CH_EXTRA, 1, 0)

    def start_gathers(idxv, gpv, gqv, sem):
        pltpu.async_copy(p_hbm.at[idxv.at[0]], gpv.at[pl.ds(0, SB)], sem)
        pltpu.async_copy(p_hbm.at[idxv.at[1]], gpv.at[pl.ds(SB, SB)], sem)
        pltpu.async_copy(q_hbm.at[idxv.at[2]], gqv.at[pl.ds(0, SB)], sem)
        pltpu.async_copy(q_hbm.at[idxv.at[3]], gqv.at[pl.ds(SB, SB)], sem)

    def drain_gathers(idxv, gpv, gqv, sem):
        pltpu.make_async_copy(p_hbm.at[idxv.at[0]],
                              gpv.at[pl.ds(0, SB)], sem).wait()
        pltpu.make_async_copy(p_hbm.at[idxv.at[1]],
                              gpv.at[pl.ds(SB, SB)], sem).wait()
        pltpu.make_async_copy(q_hbm.at[idxv.at[2]],
                              gqv.at[pl.ds(0, SB)], sem).wait()
        pltpu.make_async_copy(q_hbm.at[idxv.at[3]],
                              gqv.at[pl.ds(SB, SB)], sem).wait()

    def start_scatters(gidxv, gpv, sem):
        pltpu.async_copy(gpv.at[pl.ds(0, SB)],
                         pool_sh.at[gidxv.at[0]], sem, add=True)
        pltpu.async_copy(gpv.at[pl.ds(SB, SB)],
                         pool_sh.at[gidxv.at[1]], sem, add=True)
        pltpu.async_copy(ones_v, cnt_sh.at[gidxv.at[0]], sem, add=True)
        pltpu.async_copy(ones_v, cnt_sh.at[gidxv.at[1]], sem, add=True)

    def drain_scatters(gidxv, gpv, sem):
        pltpu.make_async_copy(gpv.at[pl.ds(0, SB)],
                              pool_sh.at[gidxv.at[0]], sem).wait()
        pltpu.make_async_copy(gpv.at[pl.ds(SB, SB)],
                              pool_sh.at[gidxv.at[1]], sem).wait()
        pltpu.make_async_copy(ones_v, cnt_sh.at[gidxv.at[0]], sem).wait()
        pltpu.make_async_copy(ones_v, cnt_sh.at[gidxv.at[1]], sem).wait()

    pltpu.sync_copy(idx_hbm.at[wid], idx_a)
    start_gathers(idx_a, gp_a, gq_a, semga)

    sets = ((gp_a, gq_a, idx_a, gidx_a, semga, semsa),
            (gp_b, gq_b, idx_b, gidx_b, semgb, semsb))

    def chunk_body(i, gpx, gqx, idxx, gidxx, semgx, semsx,
                   gpy, gqy, idxy, gidxy, semgy, semsy):
        baseh = (wid + i * NW) * CHH
        tcp = pltpu.async_copy(t_hbm.at[pl.ds(baseh, CHH)], t_v, semt)

        for half in range(2):
            for m in range(SB // 16):
                s = pl.ds(m * 16, 16)
                gidxx[half, s] = plsc.load_gather(batch_v, [idxx[half, s]])

        @pl.when(i >= 1)
        def _():
            drain_scatters(gidxy, gpy, semsy)

        @pl.when(i + 1 < nch)
        def _():
            pltpu.sync_copy(idx_hbm.at[wid + (i + 1) * NW], idxy)
            start_gathers(idxy, gpy, gqy, semgy)

        drain_gathers(idxx, gpx, gqx, semgx)
        tcp.wait()

        @plsc.parallel_loop(0, CHH, step=1, unroll=4)
        def rows(r):
            for half in range(2):
                for jj in range(4):
                    sv = pl.ds((half * 4 + jj) * 16, 16)
                    sg = pl.ds(jj * 16, 16)
                    gpx[half * SB + r, sg] = jnp.maximum(
                        t_v[r, sv] + gpx[half * SB + r, sg]
                        + gqx[half * SB + r, sg], 0.0)

        start_scatters(gidxx, gpx, semsx)

    def pair(u, _):
        for x in range(2):
            i = u * 2 + x

            @pl.when(i < nch)
            def _():
                chunk_body(i, *sets[x], *sets[1 - x])
        return 0
    lax.fori_loop(0, (CH_FULL + 2) // 2, pair, 0)

    @pl.when(nch % 2 == 1)
    def _():
        drain_scatters(gidx_a, gp_a, semsa)

    @pl.when(nch % 2 == 0)
    def _():
        drain_scatters(gidx_b, gp_b, semsb)

    plsc.subcore_barrier()

    @pl.when(sid == 0)
    def _():
        pltpu.sync_copy(pool_sh, pool_hbm.at[cid])
        pltpu.sync_copy(cnt_sh, cnt_hbm.at[cid])


_sc_final = pl.kernel(
    _sc_final_body,
    out_type=(jax.ShapeDtypeStruct((NC, G, H), _f32),
              jax.ShapeDtypeStruct((NC, G, 16), _f32)),
    mesh=_MESH,
    compiler_params=_SC_PARAMS,
    scratch_types=[
        pltpu.VMEM((CHH, 128), _f32),
        pltpu.VMEM((CH, H), _f32),
        pltpu.VMEM((CH, H), _f32),
        pltpu.VMEM((CH, H), _f32),
        pltpu.VMEM((CH, H), _f32),
        pltpu.VMEM((4, SB), jnp.int32),
        pltpu.VMEM((4, SB), jnp.int32),
        pltpu.VMEM((2, SB), jnp.int32),
        pltpu.VMEM((2, SB), jnp.int32),
        pltpu.VMEM((N,), jnp.int32),
        pltpu.VMEM((SB, 16), _f32),
        pltpu.VMEM((1, 16), _f32),
        pltpu.VMEM_SHARED((G, H), _f32),
        pltpu.VMEM_SHARED((G, 16), _f32),
        pltpu.SemaphoreType.DMA,
        pltpu.SemaphoreType.DMA,
        pltpu.SemaphoreType.DMA,
        pltpu.SemaphoreType.DMA,
        pltpu.SemaphoreType.DMA,
    ])


def kernel(x, edge_attr, params, edge_index, batch):
    src, dst = edge_index[0], edge_index[1]
    idx = jnp.stack([src[:EH].reshape(NCHUNKS, SB),
                     src[EH:].reshape(NCHUNKS, SB),
                     dst[:EH].reshape(NCHUNKS, SB),
                     dst[EH:].reshape(NCHUNKS, SB)], axis=1)

    def msplit(l):
        w = params[f'W_msg_{l}']
        return w[:H], w[H:2 * H], w[2 * H:]

    def usplit(l):
        w = params[f'W_upd_{l}']
        return w[:H], w[H:]

    def b2d(b):
        return b.reshape(1, H)

    def blockdiag2(w):
        z = jnp.zeros((H, H), _f32)
        return jnp.concatenate([jnp.concatenate([w, z], axis=1),
                                jnp.concatenate([z, w], axis=1)], axis=0)

    wa0, wb0, wc0 = msplit(0)
    h, p, q = _node0(x, params['Wn_enc'], b2d(params['bn_enc']),
                     wa0, wb0, b2d(params['b_msg_0']))
    eat = edge_attr.T
    t = _edge0(eat, eat, params['We_enc'], b2d(params['be_enc']), wc0)

    for l in range(2):
        e, aggp = _sc_layer(t, p, q, idx)
        wu1, wu2 = usplit(l)
        wa, wb, wc = msplit(l + 1)
        h, p, q = _upd(h, aggp, wu1, wu2, b2d(params[f'b_upd_{l}']),
                       wa, wb, b2d(params[f'b_msg_{l + 1}']))
        t = _tmat(e, blockdiag2(wc))

    poolp, cntp = _sc_final(t, p, q, idx, batch)
    out = _readout(poolp, cntp, params['W_r1'], b2d(params['b_r1']),
                   params['W_r2'], b2d(params['b_r2']))
    return out


# speedup vs baseline: 11.9575x; 1.0095x over previous
import jax
import jax.numpy as jnp
from jax import lax
from jax.experimental import pallas as pl
from jax.experimental.pallas import tpu as pltpu
from jax.experimental.pallas import tpu_sc as plsc

N = 10000
E = 320000
D_IN = 128
D_EDGE = 16
H = 64
OUT = 64
G = 16

NC, NS = 2, 16
NW = NC * NS
CH = 256
CHH = CH // 2
SB = 128
NCHUNKS = E // CH
CH_FULL = NCHUNKS // NW
CH_EXTRA = NCHUNKS - CH_FULL * NW
ROWS_PS = 624
ZR = 104
EB = 8000
EH = E // 2

_f32 = jnp.float32


def _mm(a, b):
    return lax.dot_general(a, b, (((1,), (0,)), ((), ())),
                           preferred_element_type=jnp.float32)


def _node0_body(x_ref, wn_ref, bn_ref, wa_ref, wb_ref, bm_ref,
                h_ref, p_ref, q_ref):
    h = jnp.maximum(_mm(x_ref[...], wn_ref[...]) + bn_ref[...], 0.0)
    h_ref[...] = h
    p_ref[...] = _mm(h, wa_ref[...]) + bm_ref[...]
    q_ref[...] = _mm(h, wb_ref[...])


def _dotT(a, b):
    return lax.dot_general(a, b, (((0,), (0,)), ((), ())),
                           preferred_element_type=jnp.float32)


def _edge0_body(lo_ref, hi_ref, we_ref, be_ref, wc_ref, t_ref):
    lo = jnp.maximum(_dotT(lo_ref[...], we_ref[...]) + be_ref[...], 0.0)
    hi = jnp.maximum(_dotT(hi_ref[...], we_ref[...]) + be_ref[...], 0.0)
    t_ref[...] = jnp.concatenate(
        [_mm(lo, wc_ref[...]), _mm(hi, wc_ref[...])], axis=1)


def _upd_body(h_ref, agg_ref, wu1_ref, wu2_ref, bu_ref,
              wa_ref, wb_ref, bm_ref, h2_ref, p_ref, q_ref):
    agg = agg_ref[0] + agg_ref[1]
    h2 = jnp.maximum(_mm(h_ref[...], wu1_ref[...])
                     + _mm(agg, wu2_ref[...]) + bu_ref[...], 0.0)
    h2_ref[...] = h2
    p_ref[...] = _mm(h2, wa_ref[...]) + bm_ref[...]
    q_ref[...] = _mm(h2, wb_ref[...])


def _t_body(e_ref, wc2_ref, t_ref):
    t_ref[...] = _mm(e_ref[...], wc2_ref[...])


def _readout_body(pp_ref, cc_ref, w1_ref, b1_ref, w2_ref, b2_ref, o_ref):
    pooled_sum = pp_ref[0] + pp_ref[1]
    counts = cc_ref[0] + cc_ref[1]
    pooled = pooled_sum / jnp.maximum(counts[:, 0:1], 1.0)
    hh = jnp.maximum(_mm(pooled, w1_ref[...]) + b1_ref[...], 0.0)
    o_ref[...] = _mm(hh, w2_ref[...]) + b2_ref[...]


def _sds(shape):
    return jax.ShapeDtypeStruct(shape, _f32)


_node0 = pl.pallas_call(
    _node0_body,
    out_shape=(_sds((N, H)), _sds((N, H)), _sds((N, H))))

_EBH = 16000

_edge0 = pl.pallas_call(
    _edge0_body,
    grid=(EH // _EBH,),
    compiler_params=pltpu.CompilerParams(fuse_transposed_lhs_in_matmul=True),
    in_specs=[
        pl.BlockSpec((D_EDGE, _EBH), lambda i: (0, i)),
        pl.BlockSpec((D_EDGE, _EBH), lambda i: (0, i + EH // _EBH)),
        pl.BlockSpec((D_EDGE, H), lambda i: (0, 0)),
        pl.BlockSpec((1, H), lambda i: (0, 0)),
        pl.BlockSpec((H, H), lambda i: (0, 0)),
    ],
    out_specs=pl.BlockSpec((_EBH, 128), lambda i: (i, 0)),
    out_shape=_sds((EH, 128)))

_upd = pl.pallas_call(
    _upd_body,
    out_shape=(_sds((N, H)), _sds((N, H)), _sds((N, H))))

_TB = 8000

_tmat = pl.pallas_call(
    _t_body,
    grid=(EH // _TB,),
    in_specs=[
        pl.BlockSpec((_TB, 128), lambda i: (i, 0)),
        pl.BlockSpec((128, 128), lambda i: (0, 0)),
    ],
    out_specs=pl.BlockSpec((_TB, 128), lambda i: (i, 0)),
    out_shape=_sds((EH, 128)))

_readout = pl.pallas_call(
    _readout_body,
    out_shape=_sds((G, OUT)))


_MESH = plsc.VectorSubcoreMesh(core_axis_name="c", subcore_axis_name="s",
                               num_cores=NC, num_subcores=NS)

_SC_PARAMS = pltpu.CompilerParams(use_tc_tiling_on_sc=False,
                                  needs_layout_passes=False)


def _zero_rows(ref, rows):
    def body(r, _):
        for jj in range(H // 16):
            ref[r, pl.ds(jj * 16, 16)] = jnp.zeros((16,), _f32)
        return 0
    lax.fori_loop(0, rows, body, 0)


def _sc_layer_body(t_hbm, p_hbm, q_hbm, idx_hbm,
                   e_hbm, agg_hbm,
                   t_v, gp_a, gp_b, gq_a, gq_b, idx_a, idx_b, zero_v, agg_sh,
                   semt, seme, semga, semgb, semsa, semsb):
    cid = lax.axis_index("c")
    sid = lax.axis_index("s")
    wid = sid * NC + cid

    nch = CH_FULL + jnp.where(wid < CH_EXTRA, 1, 0)

    def start_gathers(idxv, gpv, gqv, sem):
        pltpu.async_copy(p_hbm.at[idxv.at[0]], gpv.at[pl.ds(0, SB)], sem)
        pltpu.async_copy(p_hbm.at[idxv.at[1]], gpv.at[pl.ds(SB, SB)], sem)
        pltpu.async_copy(q_hbm.at[idxv.at[2]], gqv.at[pl.ds(0, SB)], sem)
        pltpu.async_copy(q_hbm.at[idxv.at[3]], gqv.at[pl.ds(SB, SB)], sem)

    def drain_gathers(idxv, gpv, gqv, sem):
        pltpu.make_async_copy(p_hbm.at[idxv.at[0]],
                              gpv.at[pl.ds(0, SB)], sem).wait()
        pltpu.make_async_copy(p_hbm.at[idxv.at[1]],
                              gpv.at[pl.ds(SB, SB)], sem).wait()
        pltpu.make_async_copy(q_hbm.at[idxv.at[2]],
                              gqv.at[pl.ds(0, SB)], sem).wait()
        pltpu.make_async_copy(q_hbm.at[idxv.at[3]],
                              gqv.at[pl.ds(SB, SB)], sem).wait()

    def drain_scatters(idxv, gpv, sem):
        pltpu.make_async_copy(gpv.at[pl.ds(0, SB)],
                              agg_sh.at[idxv.at[2]], sem).wait()
        pltpu.make_async_copy(gpv.at[pl.ds(SB, SB)],
                              agg_sh.at[idxv.at[3]], sem).wait()

    pltpu.sync_copy(idx_hbm.at[wid], idx_a)
    start_gathers(idx_a, gp_a, gq_a, semga)

    _zero_rows(zero_v, ZR)

    def zcp(m, _):
        pltpu.sync_copy(zero_v,
                        agg_sh.at[pl.ds(sid * ROWS_PS + m * ZR, ZR)])
        return 0
    lax.fori_loop(0, ROWS_PS // ZR, zcp, 0)

    @pl.when(sid == NS - 1)
    def _():
        pltpu.sync_copy(zero_v.at[pl.ds(0, N - NS * ROWS_PS)],
                        agg_sh.at[pl.ds(NS * ROWS_PS, N - NS * ROWS_PS)])
    plsc.subcore_barrier()

    sets = ((gp_a, gq_a, idx_a, semga, semsa),
            (gp_b, gq_b, idx_b, semgb, semsb))

    def chunk_body(i, gpx, gqx, idxx, semgx, semsx,
                   gpy, gqy, idxy, semgy, semsy):
        baseh = (wid + i * NW) * CHH

        @pl.when(i >= 1)
        def _():
            pltpu.make_async_copy(t_v, e_hbm.at[pl.ds(0, CHH)],
                                  seme).wait()
        tcp = pltpu.async_copy(t_hbm.at[pl.ds(baseh, CHH)], t_v, semt)

        @pl.when(i >= 1)
        def _():
            drain_scatters(idxy, gpy, semsy)

        @pl.when(i + 1 < nch)
        def _():
            pltpu.sync_copy(idx_hbm.at[wid + (i + 1) * NW], idxy)
            start_gathers(idxy, gpy, gqy, semgy)

        drain_gathers(idxx, gpx, gqx, semgx)
        tcp.wait()

        @plsc.parallel_loop(0, CHH, step=1, unroll=4)
        def rows(r):
            for half in range(2):
                for jj in range(4):
                    sv = pl.ds((half * 4 + jj) * 16, 16)
                    sg = pl.ds(jj * 16, 16)
                    v = jnp.maximum(
                        t_v[r, sv] + gpx[half * SB + r, sg]
                        + gqx[half * SB + r, sg], 0.0)
                    t_v[r, sv] = v
                    gpx[half * SB + r, sg] = v

        pltpu.async_copy(t_v, e_hbm.at[pl.ds(baseh, CHH)], seme)
        pltpu.async_copy(gpx.at[pl.ds(0, SB)],
                         agg_sh.at[idxx.at[2]], semsx, add=True)
        pltpu.async_copy(gpx.at[pl.ds(SB, SB)],
                         agg_sh.at[idxx.at[3]], semsx, add=True)

    def pair(u, _):
        for x in range(2):
            i = u * 2 + x

            @pl.when(i < nch)
            def _():
                chunk_body(i, *sets[x], *sets[1 - x])
        return 0
    lax.fori_loop(0, (CH_FULL + 2) // 2, pair, 0)

    pltpu.make_async_copy(t_v, e_hbm.at[pl.ds(0, CHH)], seme).wait()

    @pl.when(nch % 2 == 1)
    def _():
        drain_scatters(idx_a, gp_a, semsa)

    @pl.when(nch % 2 == 0)
    def _():
        drain_scatters(idx_b, gp_b, semsb)

    plsc.subcore_barrier()

    @pl.when(sid < NS - 1)
    def _():
        pltpu.sync_copy(agg_sh.at[pl.ds(sid * ROWS_PS, ROWS_PS)],
                        agg_hbm.at[cid].at[pl.ds(sid * ROWS_PS, ROWS_PS)])

    @pl.when(sid == NS - 1)
    def _():
        pltpu.sync_copy(
            agg_sh.at[pl.ds((NS - 1) * ROWS_PS, N - (NS - 1) * ROWS_PS)],
            agg_hbm.at[cid].at[pl.ds((NS - 1) * ROWS_PS,
                                     N - (NS - 1) * ROWS_PS)])


_sc_layer = pl.kernel(
    _sc_layer_body,
    out_type=(jax.ShapeDtypeStruct((EH, 128), _f32),
              jax.ShapeDtypeStruct((NC, N, H), _f32)),
    mesh=_MESH,
    compiler_params=_SC_PARAMS,
    scratch_types=[
        pltpu.VMEM((CHH, 128), _f32),
        pltpu.VMEM((CH, H), _f32),
        pltpu.VMEM((CH, H), _f32),
        pltpu.VMEM((CH, H), _f32),
        pltpu.VMEM((CH, H), _f32),
        pltpu.VMEM((4, SB), jnp.int32),
        pltpu.VMEM((4, SB), jnp.int32),
        pltpu.VMEM((ZR, H), _f32),
        pltpu.VMEM_SHARED((N, H), _f32),
        pltpu.SemaphoreType.DMA,
        pltpu.SemaphoreType.DMA,
        pltpu.SemaphoreType.DMA,
        pltpu.SemaphoreType.DMA,
        pltpu.SemaphoreType.DMA,
        pltpu.SemaphoreType.DMA,
    ])


def _sc_final_body(t_hbm, p_hbm, q_hbm, idx_hbm, batch_hbm,
                   pool_hbm, cnt_hbm,
                   t_v, gp_a, gp_b, gq_a, gq_b, idx_a, idx_b, gidx_a, gidx_b,
                   batch_v, ones_v, z16_v,
                   pool_sh, cnt_sh, semt, semga, semgb, semsa, semsb):
    cid = lax.axis_index("c")
    sid = lax.axis_index("s")
    wid = sid * NC + cid

    pltpu.sync_copy(batch_hbm, batch_v)

    def ones_rows(r, _):
        ones_v[r, pl.ds(0, 16)] = jnp.ones((16,), _f32)
        return 0
    lax.fori_loop(0, SB, ones_rows, 0)
    _zero_rows(gp_a, 1)
    z16_v[0, pl.ds(0, 16)] = jnp.zeros((16,), _f32)
    pltpu.sync_copy(gp_a.at[pl.ds(0, 1)], pool_sh.at[pl.ds(sid, 1)])
    pltpu.sync_copy(z16_v, cnt_sh.at[pl.ds(sid, 1)])
    plsc.subcore_barrier()

    nch = CH_FULL + jnp.where(wid < CH_EXTRA, 1, 0)

    def start_gathers(idxv, gpv, gqv, sem):
        pltpu.async_copy(p_hbm.at[idxv.at[0]], gpv.at[pl.ds(0, SB)], sem)
        pltpu.async_copy(p_hbm.at[idxv.at[1]], gpv.at[pl.ds(SB, SB)], sem)
        pltpu.async_copy(q_hbm.at[idxv.at[2]], gqv.at[pl.ds(0, SB)], sem)
        pltpu.async_copy(q_hbm.at[idxv.at[3]], gqv.at[pl.ds(SB, SB)], sem)

    def drain_gathers(idxv, gpv, gqv, sem):
        pltpu.make_async_copy(p_hbm.at[idxv.at[0]],
                              gpv.at[pl.ds(0, SB)], sem).wait()
        pltpu.make_async_copy(p_hbm.at[idxv.at[1]],
                              gpv.at[pl.ds(SB, SB)], sem).wait()
        pltpu.make_async_copy(q_hbm.at[idxv.at[2]],
                              gqv.at[pl.ds(0, SB)], sem).wait()
        pltpu.make_async_copy(q_hbm.at[idxv.at[3]],
                              gqv.at[pl.ds(SB, SB)], sem).wait()

    def start_scatters(gidxv, gpv, sem):
        pltpu.async_copy(gpv.at[pl.ds(0, SB)],
                         pool_sh.at[gidxv.at[0]], sem, add=True)
        pltpu.async_copy(gpv.at[pl.ds(SB, SB)],
                         pool_sh.at[gidxv.at[1]], sem, add=True)
        pltpu.async_copy(ones_v, cnt_sh.at[gidxv.at[0]], sem, add=True)
        pltpu.async_copy(ones_v, cnt_sh.at[gidxv.at[1]], sem, add=True)

    def drain_scatters(gidxv, gpv, sem):
        pltpu.make_async_copy(gpv.at[pl.ds(0, SB)],
                              pool_sh.at[gidxv.at[0]], sem).wait()
        pltpu.make_async_copy(gpv.at[pl.ds(SB, SB)],
                              pool_sh.at[gidxv.at[1]], sem).wait()
        pltpu.make_async_copy(ones_v, cnt_sh.at[gidxv.at[0]], sem).wait()
        pltpu.make_async_copy(ones_v, cnt_sh.at[gidxv.at[1]], sem).wait()

    pltpu.sync_copy(idx_hbm.at[wid], idx_a)
    start_gathers(idx_a, gp_a, gq_a, semga)

    sets = ((gp_a, gq_a, idx_a, gidx_a, semga, semsa),
            (gp_b, gq_b, idx_b, gidx_b, semgb, semsb))

    def chunk_body(i, gpx, gqx, idxx, gidxx, semgx, semsx,
                   gpy, gqy, idxy, gidxy, semgy, semsy):
        baseh = (wid + i * NW) * CHH
        tcp = pltpu.async_copy(t_hbm.at[pl.ds(baseh, CHH)], t_v, semt)

        for half in range(2):
            for m in range(SB // 16):
                s = pl.ds(m * 16, 16)
                gidxx[half, s] = plsc.load_gather(batch_v, [idxx[half, s]])

        @pl.when(i >= 1)
        def _():
            drain_scatters(gidxy, gpy, semsy)

        @pl.when(i + 1 < nch)
        def _():
            pltpu.sync_copy(idx_hbm.at[wid + (i + 1) * NW], idxy)
            start_gathers(idxy, gpy, gqy, semgy)

        drain_gathers(idxx, gpx, gqx, semgx)
        tcp.wait()

        @plsc.parallel_loop(0, CHH, step=1, unroll=4)
        def rows(r):
            for half in range(2):
                for jj in range(4):
                    sv = pl.ds((half * 4 + jj) * 16, 16)
                    sg = pl.ds(jj * 16, 16)
                    gpx[half * SB + r, sg] = jnp.maximum(
                        t_v[r, sv] + gpx[half * SB + r, sg]
                        + gqx[half * SB + r, sg], 0.0)

        start_scatters(gidxx, gpx, semsx)

    def pair(u, _):
        for x in range(2):
            i = u * 2 + x

            @pl.when(i < nch)
            def _():
                chunk_body(i, *sets[x], *sets[1 - x])
        return 0
    lax.fori_loop(0, (CH_FULL + 2) // 2, pair, 0)

    @pl.when(nch % 2 == 1)
    def _():
        drain_scatters(gidx_a, gp_a, semsa)

    @pl.when(nch % 2 == 0)
    def _():
        drain_scatters(gidx_b, gp_b, semsb)

    plsc.subcore_barrier()

    @pl.when(sid == 0)
    def _():
        pltpu.sync_copy(pool_sh, pool_hbm.at[cid])
        pltpu.sync_copy(cnt_sh, cnt_hbm.at[cid])


_sc_final = pl.kernel(
    _sc_final_body,
    out_type=(jax.ShapeDtypeStruct((NC, G, H), _f32),
              jax.ShapeDtypeStruct((NC, G, 16), _f32)),
    mesh=_MESH,
    compiler_params=_SC_PARAMS,
    scratch_types=[
        pltpu.VMEM((CHH, 128), _f32),
        pltpu.VMEM((CH, H), _f32),
        pltpu.VMEM((CH, H), _f32),
        pltpu.VMEM((CH, H), _f32),
        pltpu.VMEM((CH, H), _f32),
        pltpu.VMEM((4, SB), jnp.int32),
        pltpu.VMEM((4, SB), jnp.int32),
        pltpu.VMEM((2, SB), jnp.int32),
        pltpu.VMEM((2, SB), jnp.int32),
        pltpu.VMEM((N,), jnp.int32),
        pltpu.VMEM((SB, 16), _f32),
        pltpu.VMEM((1, 16), _f32),
        pltpu.VMEM_SHARED((G, H), _f32),
        pltpu.VMEM_SHARED((G, 16), _f32),
        pltpu.SemaphoreType.DMA,
        pltpu.SemaphoreType.DMA,
        pltpu.SemaphoreType.DMA,
        pltpu.SemaphoreType.DMA,
        pltpu.SemaphoreType.DMA,
    ])


def kernel(x, edge_attr, params, edge_index, batch):
    src, dst = edge_index[0], edge_index[1]
    idx = jnp.stack([src[:EH].reshape(NCHUNKS, SB),
                     src[EH:].reshape(NCHUNKS, SB),
                     dst[:EH].reshape(NCHUNKS, SB),
                     dst[EH:].reshape(NCHUNKS, SB)], axis=1)

    def msplit(l):
        w = params[f'W_msg_{l}']
        return w[:H], w[H:2 * H], w[2 * H:]

    def usplit(l):
        w = params[f'W_upd_{l}']
        return w[:H], w[H:]

    def b2d(b):
        return b.reshape(1, H)

    def blockdiag2(w):
        z = jnp.zeros((H, H), _f32)
        return jnp.concatenate([jnp.concatenate([w, z], axis=1),
                                jnp.concatenate([z, w], axis=1)], axis=0)

    wa0, wb0, wc0 = msplit(0)
    h, p, q = _node0(x, params['Wn_enc'], b2d(params['bn_enc']),
                     wa0, wb0, b2d(params['b_msg_0']))
    eat = edge_attr.T
    t = _edge0(eat, eat, params['We_enc'], b2d(params['be_enc']), wc0)

    for l in range(2):
        e, aggp = _sc_layer(t, p, q, idx)
        wu1, wu2 = usplit(l)
        wa, wb, wc = msplit(l + 1)
        h, p, q = _upd(h, aggp, wu1, wu2, b2d(params[f'b_upd_{l}']),
                       wa, wb, b2d(params[f'b_msg_{l + 1}']))
        t = _tmat(e, blockdiag2(wc))

    poolp, cntp = _sc_final(t, p, q, idx, batch)
    out = _readout(poolp, cntp, params['W_r1'], b2d(params['b_r1']),
                   params['W_r2'], b2d(params['b_r2']))
    return out
